# Initial kernel scaffold; baseline (speedup 1.0000x reference)
import functools

import jax
import jax.numpy as jnp
from jax import lax
from jax.experimental import pallas as pl
from jax.experimental.pallas import tpu as pltpu
from jax.experimental.pallas import tpu_sc as plsc

_N = 10000
_CH = 128
_CHUNKS = 81
_NW = 32
_EPAD = _NW * _CHUNKS * _CH
_AROWS = 10240
_ZROWS = _AROWS // 16


def _take(v, idx):
    return jnp.take(v, idx, mode="promise_in_bounds")


def _tc_prep1(x_ref, w1_ref, a1_ref, h_ref, asad_ref):
    h = jnp.dot(x_ref[...], w1_ref[...], preferred_element_type=jnp.float32)
    h_ref[...] = h
    asad_ref[...] = jnp.dot(h, a1_ref[...], preferred_element_type=jnp.float32)


def _tc_mid(p_ref, e64_ref, bden_ref, b1_ref, w2m_ref, tab_ref):
    pp = p_ref[0] + p_ref[1]
    pp = pp[:_N]
    msg = jnp.dot(pp, e64_ref[...], preferred_element_type=jnp.float32)
    den = jnp.dot(pp, bden_ref[...], preferred_element_type=jnp.float32)
    x1 = msg / den + b1_ref[...]
    x1 = jnp.where(x1 > 0, x1, jnp.expm1(x1))
    tab_ref[...] = jnp.dot(x1, w2m_ref[...], preferred_element_type=jnp.float32)


def _tc_out(p_ref, d16_ref, b2p_ref, o_ref):
    pp = p_ref[0, :_N] + p_ref[1, :_N]
    den = jnp.dot(pp, d16_ref[...], preferred_element_type=jnp.float32)
    o = pp / den + b2p_ref[...]
    mask = lax.broadcasted_iota(jnp.int32, (1, 16), 1) < 7
    om = jnp.where(mask, o, -1e30)
    m = jnp.max(om, axis=1, keepdims=True)
    ez = jnp.where(mask, jnp.exp(om - m), 0.0)
    lse = jnp.log(jnp.sum(ez, axis=1, keepdims=True)) + m
    o_ref[...] = o - lse


def _sc_layer1(src_hbm, dst_hbm, htab_hbm, asad_hbm, out_hbm,
               src_v, dst_v, h_blk, s_blk, d_blk, msg_blk,
               sem_h, sem_s, sem_d, accum):
    c = lax.axis_index("c")
    s = lax.axis_index("s")
    w = c * 16 + s
    iota = lax.iota(jnp.int32, 16)
    shift_idx = (iota % 8) + 8
    lane_lt8 = iota < 8
    zeros16 = jnp.zeros((16,), jnp.float32)

    def _zrow(i, _):
        for v in range(5):
            msg_blk[i, pl.ds(v * 16, 16)] = zeros16
        return 0
    lax.fori_loop(0, _CH, _zrow, 0)
    for k in range(_ZROWS // _CH):
        pltpu.sync_copy(msg_blk, accum.at[pl.ds(s * _ZROWS + k * _CH, _CH)])
    plsc.subcore_barrier()

    def _chunk(j, _):
        base = (w * _CHUNKS + j) * _CH
        pltpu.sync_copy(src_hbm.at[pl.ds(base, _CH)], src_v)
        pltpu.sync_copy(dst_hbm.at[pl.ds(base, _CH)], dst_v)
        cp_h = pltpu.async_copy(htab_hbm.at[src_v], h_blk, sem_h)
        cp_s = pltpu.async_copy(asad_hbm.at[src_v], s_blk, sem_s)
        cp_d = pltpu.async_copy(asad_hbm.at[dst_v], d_blk, sem_d)
        cp_s.wait()
        cp_d.wait()
        cp_h.wait()

        def _edge(i, _):
            srow = s_blk[i, :]
            drow = d_blk[i, :]
            e = srow + _take(drow, shift_idx)
            e = jnp.where(e > 0, e, 0.2 * e)
            ex = jnp.exp(e)
            msg_blk[i, pl.ds(64, 16)] = jnp.where(lane_lt8, ex, 0.0)
            for v in range(4):
                msg_blk[i, pl.ds(v * 16, 16)] = (
                    h_blk[i, pl.ds(v * 16, 16)] * _take(ex, 2 * v + iota // 8))
            return 0
        lax.fori_loop(0, _CH, _edge, 0)
        pltpu.sync_copy(msg_blk, accum.at[dst_v], add=True)
        return 0
    lax.fori_loop(0, _CHUNKS, _chunk, 0)
    plsc.subcore_barrier()
    for k in range(_ZROWS // _CH):
        r0 = s * _ZROWS + k * _CH
        pltpu.sync_copy(accum.at[pl.ds(r0, _CH)], out_hbm.at[c, pl.ds(r0, _CH)])


def _sc_layer2(src_hbm, dst_hbm, tab_hbm, out_hbm,
               src_v, dst_v, s_blk, d_blk, msg_blk, sem_s, sem_d, accum):
    c = lax.axis_index("c")
    s = lax.axis_index("s")
    w = c * 16 + s
    iota = lax.iota(jnp.int32, 16)
    zeros16 = jnp.zeros((16,), jnp.float32)

    def _zrow(i, _):
        msg_blk[i, :] = zeros16
        return 0
    lax.fori_loop(0, _CH, _zrow, 0)
    for k in range(_ZROWS // _CH):
        pltpu.sync_copy(msg_blk, accum.at[pl.ds(s * _ZROWS + k * _CH, _CH)])
    plsc.subcore_barrier()

    def _chunk(j, _):
        base = (w * _CHUNKS + j) * _CH
        pltpu.sync_copy(src_hbm.at[pl.ds(base, _CH)], src_v)
        pltpu.sync_copy(dst_hbm.at[pl.ds(base, _CH)], dst_v)
        cp_s = pltpu.async_copy(tab_hbm.at[src_v], s_blk, sem_s)
        cp_d = pltpu.async_copy(tab_hbm.at[dst_v], d_blk, sem_d)
        cp_s.wait()
        cp_d.wait()

        def _edge(i, _):
            srow = s_blk[i, :]
            e = s_blk[i, 7] + d_blk[i, 8]
            e = jnp.where(e > 0, e, 0.2 * e)
            ex = jnp.exp(jnp.full((16,), e, jnp.float32))
            t = jnp.where(iota == 7, 1.0, srow)
            msg_blk[i, :] = jnp.where(iota < 8, t * ex, 0.0)
            return 0
        lax.fori_loop(0, _CH, _edge, 0)
        pltpu.sync_copy(msg_blk, accum.at[dst_v], add=True)
        return 0
    lax.fori_loop(0, _CHUNKS, _chunk, 0)
    plsc.subcore_barrier()
    for k in range(_ZROWS // _CH):
        r0 = s * _ZROWS + k * _CH
        pltpu.sync_copy(accum.at[pl.ds(r0, _CH)], out_hbm.at[c, pl.ds(r0, _CH)])


def _run_sc1(srcp, dstp, h1tab, asad1):
    mesh = plsc.VectorSubcoreMesh(core_axis_name="c", subcore_axis_name="s")
    f = pl.kernel(
        _sc_layer1,
        out_type=jax.ShapeDtypeStruct((2, _AROWS, 80), jnp.float32),
        mesh=mesh,
        scratch_types=[
            pltpu.VMEM((_CH,), jnp.int32),
            pltpu.VMEM((_CH,), jnp.int32),
            pltpu.VMEM((_CH, 64), jnp.float32),
            pltpu.VMEM((_CH, 16), jnp.float32),
            pltpu.VMEM((_CH, 16), jnp.float32),
            pltpu.VMEM((_CH, 80), jnp.float32),
            pltpu.SemaphoreType.DMA,
            pltpu.SemaphoreType.DMA,
            pltpu.SemaphoreType.DMA,
            pltpu.VMEM_SHARED((_AROWS, 80), jnp.float32),
        ],
    )
    return f(srcp, dstp, h1tab, asad1)


def _run_sc2(srcp, dstp, tab2):
    mesh = plsc.VectorSubcoreMesh(core_axis_name="c", subcore_axis_name="s")
    f = pl.kernel(
        _sc_layer2,
        out_type=jax.ShapeDtypeStruct((2, _AROWS, 16), jnp.float32),
        mesh=mesh,
        scratch_types=[
            pltpu.VMEM((_CH,), jnp.int32),
            pltpu.VMEM((_CH,), jnp.int32),
            pltpu.VMEM((_CH, 16), jnp.float32),
            pltpu.VMEM((_CH, 16), jnp.float32),
            pltpu.VMEM((_CH, 16), jnp.float32),
            pltpu.SemaphoreType.DMA,
            pltpu.SemaphoreType.DMA,
            pltpu.VMEM_SHARED((_AROWS, 16), jnp.float32),
        ],
    )
    return f(srcp, dstp, tab2)


def kernel(x, edge_index, W1, a_src1, a_dst1, b1, W2, a_src2, a_dst2, b2):
    loops = jnp.arange(_N, dtype=jnp.int32)
    src = jnp.concatenate([edge_index[0], loops])
    dst = jnp.concatenate([edge_index[1], loops])
    npad = _EPAD - src.shape[0]
    srcp = jnp.concatenate([src, jnp.zeros((npad,), jnp.int32)])
    dstp = jnp.concatenate([dst, jnp.full((npad,), _N, jnp.int32)])

    eye8 = jnp.eye(8, dtype=jnp.float32)
    a1s = (eye8[:, None, :] * a_src1[:, :, None]).reshape(64, 8)
    a1d = (eye8[:, None, :] * a_dst1[:, :, None]).reshape(64, 8)
    A1 = jnp.concatenate([a1s, a1d], axis=1)

    E64 = jnp.concatenate(
        [jnp.eye(64, dtype=jnp.float32), jnp.zeros((16, 64), jnp.float32)], 0)
    BDEN = jnp.concatenate(
        [jnp.zeros((64, 64), jnp.float32),
         jnp.repeat(eye8, 8, axis=1),
         jnp.zeros((8, 64), jnp.float32)], 0)

    M = jnp.zeros((7, 16), jnp.float32)
    M = M.at[:, 0:7].set(jnp.eye(7, dtype=jnp.float32))
    M = M.at[:, 7].set(a_src2[0])
    M = M.at[:, 8].set(a_dst2[0])
    W2M = W2 @ M

    D16 = jnp.zeros((16, 16), jnp.float32).at[7, :].set(1.0)
    b2p = jnp.concatenate([b2, jnp.zeros((9,), jnp.float32)])

    h1tab, asad1 = pl.pallas_call(
        _tc_prep1,
        out_shape=(jax.ShapeDtypeStruct((_N, 64), jnp.float32),
                   jax.ShapeDtypeStruct((_N, 16), jnp.float32)),
    )(x, W1, A1)

    part1 = _run_sc1(srcp, dstp, h1tab, asad1)

    tab2 = pl.pallas_call(
        _tc_mid,
        out_shape=jax.ShapeDtypeStruct((_N, 16), jnp.float32),
    )(part1, E64, BDEN, b1, W2M)

    part2 = _run_sc2(srcp, dstp, tab2)

    out16 = pl.pallas_call(
        _tc_out,
        out_shape=jax.ShapeDtypeStruct((_N, 16), jnp.float32),
    )(part2, D16, b2p)

    return out16[:, :7]


# speedup vs baseline: 51.6757x; 51.6757x over previous
import functools

import jax
import jax.numpy as jnp
from jax import lax
from jax.experimental import pallas as pl
from jax.experimental.pallas import tpu as pltpu
from jax.experimental.pallas import tpu_sc as plsc

_N = 10000
_CH = 128
_CHUNKS = 81
_NW = 32
_EPAD = _NW * _CHUNKS * _CH
_AROWS = 10240
_ZROWS = _AROWS // 16


def _tc_prep1(x_ref, w1_ref, a1_ref, a1r_ref, h_ref, asad_ref, adas_ref):
    h = jnp.dot(x_ref[...], w1_ref[...], preferred_element_type=jnp.float32)
    h_ref[...] = h
    asad_ref[...] = jnp.dot(h, a1_ref[...], preferred_element_type=jnp.float32)
    adas_ref[...] = jnp.dot(h, a1r_ref[...], preferred_element_type=jnp.float32)


def _tc_mid(p_ref, e64_ref, bden_ref, b1_ref, w2m_ref, tab_ref):
    pp = p_ref[0] + p_ref[1]
    pp = pp[:_N]
    msg = jnp.dot(pp, e64_ref[...], preferred_element_type=jnp.float32)
    den = jnp.dot(pp, bden_ref[...], preferred_element_type=jnp.float32)
    x1 = msg / den + b1_ref[...]
    x1 = jnp.where(x1 > 0, x1, jnp.exp(jnp.minimum(x1, 0.0)) - 1.0)
    tab_ref[...] = jnp.dot(x1, w2m_ref[...], preferred_element_type=jnp.float32)


def _tc_out(p_ref, d16_ref, b2p_ref, o_ref):
    pp = p_ref[0, :_N] + p_ref[1, :_N]
    den = jnp.dot(pp, d16_ref[...], preferred_element_type=jnp.float32)
    o = pp / den + b2p_ref[...]
    mask = lax.broadcasted_iota(jnp.int32, (1, 16), 1) < 7
    om = jnp.where(mask, o, -1e30)
    m = jnp.max(om, axis=1, keepdims=True)
    ez = jnp.where(mask, jnp.exp(om - m), 0.0)
    lse = jnp.log(jnp.sum(ez, axis=1, keepdims=True)) + m
    o_ref[...] = o - lse


def _sc_layer1(src_hbm, dst_hbm, htab_hbm, asad_hbm, adas_hbm, out_hbm,
               src_v, dst_v, h_blk, s_blk, d_blk, msg_blk, ex_s,
               sem_h, sem_s, sem_d, accum):
    c = lax.axis_index("c")
    s = lax.axis_index("s")
    w = c * 16 + s
    iota = lax.iota(jnp.int32, 16)
    lane_lt8 = iota < 8
    zeros16 = jnp.zeros((16,), jnp.float32)

    def _zrow(i, _):
        for v in range(5):
            msg_blk[i, pl.ds(v * 16, 16)] = zeros16
        return 0
    lax.fori_loop(0, _CH, _zrow, 0)
    for k in range(_ZROWS // _CH):
        pltpu.sync_copy(msg_blk, accum.at[pl.ds(s * _ZROWS + k * _CH, _CH)])
    plsc.subcore_barrier()

    def _chunk(j, _):
        base = (w * _CHUNKS + j) * _CH
        pltpu.sync_copy(src_hbm.at[pl.ds(base, _CH)], src_v)
        pltpu.sync_copy(dst_hbm.at[pl.ds(base, _CH)], dst_v)
        cp_h = pltpu.async_copy(htab_hbm.at[src_v], h_blk, sem_h)
        cp_s = pltpu.async_copy(asad_hbm.at[src_v], s_blk, sem_s)
        cp_d = pltpu.async_copy(adas_hbm.at[dst_v], d_blk, sem_d)
        cp_s.wait()
        cp_d.wait()
        cp_h.wait()

        def _edge(i, _):
            e = s_blk[i, :] + d_blk[i, :]
            e = jnp.where(e > 0, e, 0.2 * e)
            ex = jnp.exp(e)
            ex_s[...] = ex
            msg_blk[i, pl.ds(64, 16)] = jnp.where(lane_lt8, ex, 0.0)
            for v in range(4):
                av = plsc.load_gather(ex_s, [2 * v + iota // 8])
                msg_blk[i, pl.ds(v * 16, 16)] = h_blk[i, pl.ds(v * 16, 16)] * av
            return 0
        lax.fori_loop(0, _CH, _edge, 0)
        pltpu.sync_copy(msg_blk, accum.at[dst_v], add=True)
        return 0
    lax.fori_loop(0, _CHUNKS, _chunk, 0)
    plsc.subcore_barrier()
    for k in range(_ZROWS // _CH):
        r0 = s * _ZROWS + k * _CH
        pltpu.sync_copy(accum.at[pl.ds(r0, _CH)], out_hbm.at[c, pl.ds(r0, _CH)])


def _sc_layer2(src_hbm, dst_hbm, tab_hbm, out_hbm,
               src_v, dst_v, s_blk, d_blk, msg_blk, sem_s, sem_d, accum):
    c = lax.axis_index("c")
    s = lax.axis_index("s")
    w = c * 16 + s
    iota = lax.iota(jnp.int32, 16)
    zeros16 = jnp.zeros((16,), jnp.float32)

    def _zrow(i, _):
        msg_blk[i, :] = zeros16
        return 0
    lax.fori_loop(0, _CH, _zrow, 0)
    for k in range(_ZROWS // _CH):
        pltpu.sync_copy(msg_blk, accum.at[pl.ds(s * _ZROWS + k * _CH, _CH)])
    plsc.subcore_barrier()

    def _chunk(j, _):
        base = (w * _CHUNKS + j) * _CH
        pltpu.sync_copy(src_hbm.at[pl.ds(base, _CH)], src_v)
        pltpu.sync_copy(dst_hbm.at[pl.ds(base, _CH)], dst_v)
        cp_s = pltpu.async_copy(tab_hbm.at[src_v], s_blk, sem_s)
        cp_d = pltpu.async_copy(tab_hbm.at[dst_v], d_blk, sem_d)
        cp_s.wait()
        cp_d.wait()

        def _edge(i, _):
            srow = s_blk[i, :]
            bi = iota * 0 + i
            e = (plsc.load_gather(s_blk, [bi, iota * 0 + 7])
                 + plsc.load_gather(d_blk, [bi, iota * 0 + 8]))
            e = jnp.where(e > 0, e, 0.2 * e)
            ex = jnp.exp(e)
            t = jnp.where(iota == 7, 1.0, srow)
            msg_blk[i, :] = jnp.where(iota < 8, t * ex, 0.0)
            return 0
        lax.fori_loop(0, _CH, _edge, 0)
        pltpu.sync_copy(msg_blk, accum.at[dst_v], add=True)
        return 0
    lax.fori_loop(0, _CHUNKS, _chunk, 0)
    plsc.subcore_barrier()
    for k in range(_ZROWS // _CH):
        r0 = s * _ZROWS + k * _CH
        pltpu.sync_copy(accum.at[pl.ds(r0, _CH)], out_hbm.at[c, pl.ds(r0, _CH)])


def _run_sc1(srcp, dstp, h1tab, asad1, adas1):
    mesh = plsc.VectorSubcoreMesh(core_axis_name="c", subcore_axis_name="s")
    f = pl.kernel(
        _sc_layer1,
        out_type=jax.ShapeDtypeStruct((2, _AROWS, 80), jnp.float32),
        mesh=mesh,
        scratch_types=[
            pltpu.VMEM((_CH,), jnp.int32),
            pltpu.VMEM((_CH,), jnp.int32),
            pltpu.VMEM((_CH, 64), jnp.float32),
            pltpu.VMEM((_CH, 16), jnp.float32),
            pltpu.VMEM((_CH, 16), jnp.float32),
            pltpu.VMEM((_CH, 80), jnp.float32),
            pltpu.VMEM((16,), jnp.float32),
            pltpu.SemaphoreType.DMA,
            pltpu.SemaphoreType.DMA,
            pltpu.SemaphoreType.DMA,
            pltpu.VMEM_SHARED((_AROWS, 80), jnp.float32),
        ],
        compiler_params=pltpu.CompilerParams(
            use_tc_tiling_on_sc=False, needs_layout_passes=False),
    )
    return f(srcp, dstp, h1tab, asad1, adas1)


def _run_sc2(srcp, dstp, tab2):
    mesh = plsc.VectorSubcoreMesh(core_axis_name="c", subcore_axis_name="s")
    f = pl.kernel(
        _sc_layer2,
        out_type=jax.ShapeDtypeStruct((2, _AROWS, 16), jnp.float32),
        mesh=mesh,
        scratch_types=[
            pltpu.VMEM((_CH,), jnp.int32),
            pltpu.VMEM((_CH,), jnp.int32),
            pltpu.VMEM((_CH, 16), jnp.float32),
            pltpu.VMEM((_CH, 16), jnp.float32),
            pltpu.VMEM((_CH, 16), jnp.float32),
            pltpu.SemaphoreType.DMA,
            pltpu.SemaphoreType.DMA,
            pltpu.VMEM_SHARED((_AROWS, 16), jnp.float32),
        ],
        compiler_params=pltpu.CompilerParams(
            use_tc_tiling_on_sc=False, needs_layout_passes=False),
    )
    return f(srcp, dstp, tab2)


def kernel(x, edge_index, W1, a_src1, a_dst1, b1, W2, a_src2, a_dst2, b2):
    loops = jnp.arange(_N, dtype=jnp.int32)
    src = jnp.concatenate([edge_index[0], loops])
    dst = jnp.concatenate([edge_index[1], loops])
    npad = _EPAD - src.shape[0]
    srcp = jnp.concatenate([src, jnp.zeros((npad,), jnp.int32)])
    dstp = jnp.concatenate([dst, jnp.full((npad,), _N, jnp.int32)])

    eye8 = jnp.eye(8, dtype=jnp.float32)
    a1s = (eye8[:, None, :] * a_src1[:, :, None]).reshape(64, 8)
    a1d = (eye8[:, None, :] * a_dst1[:, :, None]).reshape(64, 8)
    A1 = jnp.concatenate([a1s, a1d], axis=1)
    A1r = jnp.concatenate([a1d, a1s], axis=1)

    E64 = jnp.concatenate(
        [jnp.eye(64, dtype=jnp.float32), jnp.zeros((16, 64), jnp.float32)], 0)
    BDEN = jnp.concatenate(
        [jnp.zeros((64, 64), jnp.float32),
         jnp.repeat(eye8, 8, axis=1),
         jnp.zeros((8, 64), jnp.float32)], 0)

    M = jnp.zeros((7, 16), jnp.float32)
    M = M.at[:, 0:7].set(jnp.eye(7, dtype=jnp.float32))
    M = M.at[:, 7].set(a_src2[0])
    M = M.at[:, 8].set(a_dst2[0])
    W2M = W2 @ M

    D16 = jnp.zeros((16, 16), jnp.float32).at[7, :].set(1.0)
    b2p = jnp.concatenate([b2, jnp.zeros((9,), jnp.float32)])

    h1tab, asad1, adas1 = pl.pallas_call(
        _tc_prep1,
        out_shape=(jax.ShapeDtypeStruct((_N, 64), jnp.float32),
                   jax.ShapeDtypeStruct((_N, 16), jnp.float32),
                   jax.ShapeDtypeStruct((_N, 16), jnp.float32)),
    )(x, W1, A1, A1r)

    part1 = _run_sc1(srcp, dstp, h1tab, asad1, adas1)

    tab2 = pl.pallas_call(
        _tc_mid,
        out_shape=jax.ShapeDtypeStruct((_N, 16), jnp.float32),
    )(part1, E64, BDEN, b1, W2M)

    part2 = _run_sc2(srcp, dstp, tab2)

    out16 = pl.pallas_call(
        _tc_out,
        out_shape=jax.ShapeDtypeStruct((_N, 16), jnp.float32),
    )(part2, D16, b2p)

    return out16[:, :7]


# speedup vs baseline: 52.3478x; 1.0130x over previous
import functools

import jax
import jax.numpy as jnp
from jax import lax
from jax.experimental import pallas as pl
from jax.experimental.pallas import tpu as pltpu
from jax.experimental.pallas import tpu_sc as plsc

_N = 10000
_CH = 128
_CHUNKS = 81
_NW = 32
_EPAD = _NW * _CHUNKS * _CH
_AROWS = 10240
_ZROWS = _AROWS // 16


def _tc_prep1(x_ref, w1_ref, a1_ref, a1r_ref, h_ref, asad_ref, adas_ref):
    h = jnp.dot(x_ref[...], w1_ref[...], preferred_element_type=jnp.float32)
    h_ref[...] = h
    asad_ref[...] = jnp.dot(h, a1_ref[...], preferred_element_type=jnp.float32)
    adas_ref[...] = jnp.dot(h, a1r_ref[...], preferred_element_type=jnp.float32)


def _tc_mid(p_ref, e64_ref, bden_ref, b1_ref, w2m_ref, tab_ref):
    pp = p_ref[0] + p_ref[1]
    pp = pp[:_N]
    msg = jnp.dot(pp, e64_ref[...], preferred_element_type=jnp.float32)
    den = jnp.dot(pp, bden_ref[...], preferred_element_type=jnp.float32)
    x1 = msg / den + b1_ref[...]
    x1 = jnp.where(x1 > 0, x1, jnp.exp(jnp.minimum(x1, 0.0)) - 1.0)
    tab_ref[...] = jnp.dot(x1, w2m_ref[...], preferred_element_type=jnp.float32)


def _tc_out(p_ref, d16_ref, b2p_ref, o_ref):
    pp = p_ref[0, :_N] + p_ref[1, :_N]
    den = jnp.dot(pp, d16_ref[...], preferred_element_type=jnp.float32)
    o = pp / den + b2p_ref[...]
    mask = lax.broadcasted_iota(jnp.int32, (1, 16), 1) < 7
    om = jnp.where(mask, o, -1e30)
    m = jnp.max(om, axis=1, keepdims=True)
    ez = jnp.where(mask, jnp.exp(om - m), 0.0)
    lse = jnp.log(jnp.sum(ez, axis=1, keepdims=True)) + m
    o_ref[...] = o - lse


def _sc_layer1(src_hbm, dst_hbm, htab_hbm, asad_hbm, adas_hbm, out_hbm,
               src_v, dst_v, h_blk, s_blk, d_blk, msg_blk, ex_s,
               sem_h, sem_s, sem_d, accum):
    c = lax.axis_index("c")
    s = lax.axis_index("s")
    w = c * 16 + s
    iota = lax.iota(jnp.int32, 16)
    lane_lt8 = iota < 8
    zeros16 = jnp.zeros((16,), jnp.float32)

    def _zrow(i, _):
        for v in range(5):
            msg_blk[i, pl.ds(v * 16, 16)] = zeros16
        return 0
    lax.fori_loop(0, _CH, _zrow, 0)
    for k in range(_ZROWS // _CH):
        pltpu.sync_copy(msg_blk, accum.at[pl.ds(s * _ZROWS + k * _CH, _CH)])
    plsc.subcore_barrier()

    def _chunk(j, _):
        base = (w * _CHUNKS + j) * _CH
        pltpu.sync_copy(src_hbm.at[pl.ds(base, _CH)], src_v)
        pltpu.sync_copy(dst_hbm.at[pl.ds(base, _CH)], dst_v)
        cp_h = pltpu.async_copy(htab_hbm.at[src_v], h_blk, sem_h)
        cp_s = pltpu.async_copy(asad_hbm.at[src_v], s_blk, sem_s)
        cp_d = pltpu.async_copy(adas_hbm.at[dst_v], d_blk, sem_d)
        cp_s.wait()
        cp_d.wait()
        cp_h.wait()

        def _edge(i0, _):
            for u in range(4):
                i = i0 * 4 + u
                e = s_blk[i, :] + d_blk[i, :]
                e = jnp.where(e > 0, e, 0.2 * e)
                ex = jnp.exp(e)
                ex_s[u, :] = ex
                msg_blk[i, pl.ds(64, 16)] = jnp.where(lane_lt8, ex, 0.0)
            for u in range(4):
                i = i0 * 4 + u
                bu = iota * 0 + u
                for v in range(4):
                    av = plsc.load_gather(ex_s, [bu, 2 * v + iota // 8])
                    msg_blk[i, pl.ds(v * 16, 16)] = (
                        h_blk[i, pl.ds(v * 16, 16)] * av)
            return 0
        lax.fori_loop(0, _CH // 4, _edge, 0)
        pltpu.sync_copy(msg_blk, accum.at[dst_v], add=True)
        return 0
    lax.fori_loop(0, _CHUNKS, _chunk, 0)
    plsc.subcore_barrier()
    for k in range(_ZROWS // _CH):
        r0 = s * _ZROWS + k * _CH
        pltpu.sync_copy(accum.at[pl.ds(r0, _CH)], out_hbm.at[c, pl.ds(r0, _CH)])


def _sc_layer2(src_hbm, dst_hbm, tab_hbm, out_hbm,
               src_v, dst_v, s_blk, d_blk, msg_blk, sem_s, sem_d, accum):
    c = lax.axis_index("c")
    s = lax.axis_index("s")
    w = c * 16 + s
    iota = lax.iota(jnp.int32, 16)
    zeros16 = jnp.zeros((16,), jnp.float32)

    def _zrow(i, _):
        msg_blk[i, :] = zeros16
        return 0
    lax.fori_loop(0, _CH, _zrow, 0)
    for k in range(_ZROWS // _CH):
        pltpu.sync_copy(msg_blk, accum.at[pl.ds(s * _ZROWS + k * _CH, _CH)])
    plsc.subcore_barrier()

    def _chunk(j, _):
        base = (w * _CHUNKS + j) * _CH
        pltpu.sync_copy(src_hbm.at[pl.ds(base, _CH)], src_v)
        pltpu.sync_copy(dst_hbm.at[pl.ds(base, _CH)], dst_v)
        cp_s = pltpu.async_copy(tab_hbm.at[src_v], s_blk, sem_s)
        cp_d = pltpu.async_copy(tab_hbm.at[dst_v], d_blk, sem_d)
        cp_s.wait()
        cp_d.wait()

        def _edge(i0, _):
            for u in range(4):
                i = i0 * 4 + u
                srow = s_blk[i, :]
                bi = iota * 0 + i
                e = (plsc.load_gather(s_blk, [bi, iota * 0 + 7])
                     + plsc.load_gather(d_blk, [bi, iota * 0 + 8]))
                e = jnp.where(e > 0, e, 0.2 * e)
                ex = jnp.exp(e)
                t = jnp.where(iota == 7, 1.0, srow)
                msg_blk[i, :] = jnp.where(iota < 8, t * ex, 0.0)
            return 0
        lax.fori_loop(0, _CH // 4, _edge, 0)
        pltpu.sync_copy(msg_blk, accum.at[dst_v], add=True)
        return 0
    lax.fori_loop(0, _CHUNKS, _chunk, 0)
    plsc.subcore_barrier()
    for k in range(_ZROWS // _CH):
        r0 = s * _ZROWS + k * _CH
        pltpu.sync_copy(accum.at[pl.ds(r0, _CH)], out_hbm.at[c, pl.ds(r0, _CH)])


def _run_sc1(srcp, dstp, h1tab, asad1, adas1):
    mesh = plsc.VectorSubcoreMesh(core_axis_name="c", subcore_axis_name="s")
    f = pl.kernel(
        _sc_layer1,
        out_type=jax.ShapeDtypeStruct((2, _AROWS, 80), jnp.float32),
        mesh=mesh,
        scratch_types=[
            pltpu.VMEM((_CH,), jnp.int32),
            pltpu.VMEM((_CH,), jnp.int32),
            pltpu.VMEM((_CH, 64), jnp.float32),
            pltpu.VMEM((_CH, 16), jnp.float32),
            pltpu.VMEM((_CH, 16), jnp.float32),
            pltpu.VMEM((_CH, 80), jnp.float32),
            pltpu.VMEM((4, 16), jnp.float32),
            pltpu.SemaphoreType.DMA,
            pltpu.SemaphoreType.DMA,
            pltpu.SemaphoreType.DMA,
            pltpu.VMEM_SHARED((_AROWS, 80), jnp.float32),
        ],
        compiler_params=pltpu.CompilerParams(
            use_tc_tiling_on_sc=False, needs_layout_passes=False),
    )
    return f(srcp, dstp, h1tab, asad1, adas1)


def _run_sc2(srcp, dstp, tab2):
    mesh = plsc.VectorSubcoreMesh(core_axis_name="c", subcore_axis_name="s")
    f = pl.kernel(
        _sc_layer2,
        out_type=jax.ShapeDtypeStruct((2, _AROWS, 16), jnp.float32),
        mesh=mesh,
        scratch_types=[
            pltpu.VMEM((_CH,), jnp.int32),
            pltpu.VMEM((_CH,), jnp.int32),
            pltpu.VMEM((_CH, 16), jnp.float32),
            pltpu.VMEM((_CH, 16), jnp.float32),
            pltpu.VMEM((_CH, 16), jnp.float32),
            pltpu.SemaphoreType.DMA,
            pltpu.SemaphoreType.DMA,
            pltpu.VMEM_SHARED((_AROWS, 16), jnp.float32),
        ],
        compiler_params=pltpu.CompilerParams(
            use_tc_tiling_on_sc=False, needs_layout_passes=False),
    )
    return f(srcp, dstp, tab2)


def kernel(x, edge_index, W1, a_src1, a_dst1, b1, W2, a_src2, a_dst2, b2):
    loops = jnp.arange(_N, dtype=jnp.int32)
    src = jnp.concatenate([edge_index[0], loops])
    dst = jnp.concatenate([edge_index[1], loops])
    npad = _EPAD - src.shape[0]
    srcp = jnp.concatenate([src, jnp.zeros((npad,), jnp.int32)])
    dstp = jnp.concatenate([dst, jnp.full((npad,), _N, jnp.int32)])

    eye8 = jnp.eye(8, dtype=jnp.float32)
    a1s = (eye8[:, None, :] * a_src1[:, :, None]).reshape(64, 8)
    a1d = (eye8[:, None, :] * a_dst1[:, :, None]).reshape(64, 8)
    A1 = jnp.concatenate([a1s, a1d], axis=1)
    A1r = jnp.concatenate([a1d, a1s], axis=1)

    E64 = jnp.concatenate(
        [jnp.eye(64, dtype=jnp.float32), jnp.zeros((16, 64), jnp.float32)], 0)
    BDEN = jnp.concatenate(
        [jnp.zeros((64, 64), jnp.float32),
         jnp.repeat(eye8, 8, axis=1),
         jnp.zeros((8, 64), jnp.float32)], 0)

    M = jnp.zeros((7, 16), jnp.float32)
    M = M.at[:, 0:7].set(jnp.eye(7, dtype=jnp.float32))
    M = M.at[:, 7].set(a_src2[0])
    M = M.at[:, 8].set(a_dst2[0])
    W2M = W2 @ M

    D16 = jnp.zeros((16, 16), jnp.float32).at[7, :].set(1.0)
    b2p = jnp.concatenate([b2, jnp.zeros((9,), jnp.float32)])

    h1tab, asad1, adas1 = pl.pallas_call(
        _tc_prep1,
        out_shape=(jax.ShapeDtypeStruct((_N, 64), jnp.float32),
                   jax.ShapeDtypeStruct((_N, 16), jnp.float32),
                   jax.ShapeDtypeStruct((_N, 16), jnp.float32)),
    )(x, W1, A1, A1r)

    part1 = _run_sc1(srcp, dstp, h1tab, asad1, adas1)

    tab2 = pl.pallas_call(
        _tc_mid,
        out_shape=jax.ShapeDtypeStruct((_N, 16), jnp.float32),
    )(part1, E64, BDEN, b1, W2M)

    part2 = _run_sc2(srcp, dstp, tab2)

    out16 = pl.pallas_call(
        _tc_out,
        out_shape=jax.ShapeDtypeStruct((_N, 16), jnp.float32),
    )(part2, D16, b2p)

    return out16[:, :7]


# speedup vs baseline: 68.0940x; 1.3008x over previous
import functools

import jax
import jax.numpy as jnp
from jax import lax
from jax.experimental import pallas as pl
from jax.experimental.pallas import tpu as pltpu
from jax.experimental.pallas import tpu_sc as plsc

_N = 10000
_CH = 128
_CHUNKS = 81
_NW = 32
_EPAD = _NW * _CHUNKS * _CH
_AROWS = 10240
_ZROWS = _AROWS // 16


def _tc_prep1(x_ref, w1_ref, a1_ref, a1r_ref, h_ref, asad_ref, adas_ref):
    h = jnp.dot(x_ref[...], w1_ref[...], preferred_element_type=jnp.float32)
    h_ref[...] = h
    asad_ref[...] = jnp.dot(h, a1_ref[...], preferred_element_type=jnp.float32)
    adas_ref[...] = jnp.dot(h, a1r_ref[...], preferred_element_type=jnp.float32)


def _tc_mid(p_ref, e64_ref, bden_ref, b1_ref, w2m_ref, tab_ref):
    pp = p_ref[0] + p_ref[1]
    pp = pp[:_N]
    msg = jnp.dot(pp, e64_ref[...], preferred_element_type=jnp.float32)
    den = jnp.dot(pp, bden_ref[...], preferred_element_type=jnp.float32)
    x1 = msg / den + b1_ref[...]
    x1 = jnp.where(x1 > 0, x1, jnp.exp(jnp.minimum(x1, 0.0)) - 1.0)
    tab_ref[...] = jnp.dot(x1, w2m_ref[...], preferred_element_type=jnp.float32)


def _tc_out(p_ref, d16_ref, b2p_ref, o_ref):
    pp = p_ref[0, :_N] + p_ref[1, :_N]
    den = jnp.dot(pp, d16_ref[...], preferred_element_type=jnp.float32)
    o = pp / den + b2p_ref[...]
    mask = lax.broadcasted_iota(jnp.int32, (1, 16), 1) < 7
    om = jnp.where(mask, o, -1e30)
    m = jnp.max(om, axis=1, keepdims=True)
    ez = jnp.where(mask, jnp.exp(om - m), 0.0)
    lse = jnp.log(jnp.sum(ez, axis=1, keepdims=True)) + m
    o_ref[...] = o - lse


def _fill_i32(ref_row, iota, val):
    for k in range(8):
        ref_row[pl.ds(k * 16, 16)] = iota * 0 + val


def _sc_layer1(sd_hbm, htab_hbm, asad_hbm, adas_hbm, out_hbm,
               idx0, idx1, h0, h1, s0, s1, d0, d1, m0, m1, ex_s,
               sh0, sh1, ss0, ss1, sd0, sd1, sc0, sc1, accum):
    c = lax.axis_index("c")
    s = lax.axis_index("s")
    w = c * 16 + s
    iota = lax.iota(jnp.int32, 16)
    lane_lt8 = iota < 8
    zeros16 = jnp.zeros((16,), jnp.float32)
    B0 = (idx0, h0, s0, d0, m0, sh0, ss0, sd0, sc0)
    B1 = (idx1, h1, s1, d1, m1, sh1, ss1, sd1, sc1)

    def _zrow(i, _):
        for v in range(5):
            m0[i, pl.ds(v * 16, 16)] = zeros16
            m1[i, pl.ds(v * 16, 16)] = zeros16
        return 0
    lax.fori_loop(0, _CH, _zrow, 0)
    for k in range(_ZROWS // _CH):
        pltpu.sync_copy(m0, accum.at[pl.ds(s * _ZROWS + k * _CH, _CH)])
    plsc.subcore_barrier()

    def scatter_issue(B):
        idx, hb, sb, db, mb, sh, ss, sd, sc = B
        pltpu.async_copy(mb, accum.at[idx.at[1]], sc, add=True)

    def scatter_wait(B):
        idx, hb, sb, db, mb, sh, ss, sd, sc = B
        pltpu.make_async_copy(mb, accum.at[idx.at[1]], sc).wait()

    def load_issue(chunk, B):
        idx, hb, sb, db, mb, sh, ss, sd, sc = B
        scatter_wait(B)
        pltpu.sync_copy(sd_hbm.at[chunk], idx)
        pltpu.async_copy(htab_hbm.at[idx.at[0]], hb, sh)
        pltpu.async_copy(asad_hbm.at[idx.at[0]], sb, ss)
        pltpu.async_copy(adas_hbm.at[idx.at[1]], db, sd)

    def process(B):
        idx, hb, sb, db, mb, sh, ss, sd, sc = B
        pltpu.make_async_copy(asad_hbm.at[idx.at[0]], sb, ss).wait()
        pltpu.make_async_copy(adas_hbm.at[idx.at[1]], db, sd).wait()
        pltpu.make_async_copy(htab_hbm.at[idx.at[0]], hb, sh).wait()

        def _edge(i0, _):
            for u in range(4):
                i = i0 * 4 + u
                e = sb[i, :] + db[i, :]
                e = jnp.where(e > 0, e, 0.2 * e)
                ex = jnp.exp(e)
                ex_s[u, :] = ex
                mb[i, pl.ds(64, 16)] = jnp.where(lane_lt8, ex, 0.0)
            for u in range(4):
                i = i0 * 4 + u
                bu = iota * 0 + u
                for v in range(4):
                    av = plsc.load_gather(ex_s, [bu, 2 * v + iota // 8])
                    mb[i, pl.ds(v * 16, 16)] = hb[i, pl.ds(v * 16, 16)] * av
            return 0
        lax.fori_loop(0, _CH // 4, _edge, 0)
        scatter_issue(B)

    for B in (B0, B1):
        _fill_i32(B[0].at[1], iota, _N)
        scatter_issue(B)

    base = w * _CHUNKS
    load_issue(base, B0)

    def _pair(t, _):
        j0 = base + 2 * t
        load_issue(j0 + 1, B1)
        process(B0)
        load_issue(j0 + 2, B0)
        process(B1)
        return 0
    lax.fori_loop(0, (_CHUNKS - 1) // 2, _pair, 0)
    process(B0)
    scatter_wait(B0)
    scatter_wait(B1)

    plsc.subcore_barrier()
    for k in range(_ZROWS // _CH):
        r0 = s * _ZROWS + k * _CH
        pltpu.sync_copy(accum.at[pl.ds(r0, _CH)], out_hbm.at[c, pl.ds(r0, _CH)])


def _sc_layer2(sd_hbm, tab_hbm, out_hbm,
               idx0, idx1, s0, s1, d0, d1, m0, m1,
               ss0, ss1, sd0, sd1, sc0, sc1, accum):
    c = lax.axis_index("c")
    s = lax.axis_index("s")
    w = c * 16 + s
    iota = lax.iota(jnp.int32, 16)
    zeros16 = jnp.zeros((16,), jnp.float32)
    B0 = (idx0, s0, d0, m0, ss0, sd0, sc0)
    B1 = (idx1, s1, d1, m1, ss1, sd1, sc1)

    def _zrow(i, _):
        m0[i, :] = zeros16
        m1[i, :] = zeros16
        return 0
    lax.fori_loop(0, _CH, _zrow, 0)
    for k in range(_ZROWS // _CH):
        pltpu.sync_copy(m0, accum.at[pl.ds(s * _ZROWS + k * _CH, _CH)])
    plsc.subcore_barrier()

    def scatter_issue(B):
        idx, sb, db, mb, ss, sd, sc = B
        pltpu.async_copy(mb, accum.at[idx.at[1]], sc, add=True)

    def scatter_wait(B):
        idx, sb, db, mb, ss, sd, sc = B
        pltpu.make_async_copy(mb, accum.at[idx.at[1]], sc).wait()

    def load_issue(chunk, B):
        idx, sb, db, mb, ss, sd, sc = B
        scatter_wait(B)
        pltpu.sync_copy(sd_hbm.at[chunk], idx)
        pltpu.async_copy(tab_hbm.at[idx.at[0]], sb, ss)
        pltpu.async_copy(tab_hbm.at[idx.at[1]], db, sd)

    def process(B):
        idx, sb, db, mb, ss, sd, sc = B
        pltpu.make_async_copy(tab_hbm.at[idx.at[0]], sb, ss).wait()
        pltpu.make_async_copy(tab_hbm.at[idx.at[1]], db, sd).wait()

        def _edge(i0, _):
            for u in range(4):
                i = i0 * 4 + u
                srow = sb[i, :]
                bi = iota * 0 + i
                e = (plsc.load_gather(sb, [bi, iota * 0 + 7])
                     + plsc.load_gather(db, [bi, iota * 0 + 8]))
                e = jnp.where(e > 0, e, 0.2 * e)
                ex = jnp.exp(e)
                t = jnp.where(iota == 7, 1.0, srow)
                mb[i, :] = jnp.where(iota < 8, t * ex, 0.0)
            return 0
        lax.fori_loop(0, _CH // 4, _edge, 0)
        scatter_issue(B)

    for B in (B0, B1):
        _fill_i32(B[0].at[1], iota, _N)
        scatter_issue(B)

    base = w * _CHUNKS
    load_issue(base, B0)

    def _pair(t, _):
        j0 = base + 2 * t
        load_issue(j0 + 1, B1)
        process(B0)
        load_issue(j0 + 2, B0)
        process(B1)
        return 0
    lax.fori_loop(0, (_CHUNKS - 1) // 2, _pair, 0)
    process(B0)
    scatter_wait(B0)
    scatter_wait(B1)

    plsc.subcore_barrier()
    for k in range(_ZROWS // _CH):
        r0 = s * _ZROWS + k * _CH
        pltpu.sync_copy(accum.at[pl.ds(r0, _CH)], out_hbm.at[c, pl.ds(r0, _CH)])


def _run_sc1(sd, h1tab, asad1, adas1):
    mesh = plsc.VectorSubcoreMesh(core_axis_name="c", subcore_axis_name="s")
    f = pl.kernel(
        _sc_layer1,
        out_type=jax.ShapeDtypeStruct((2, _AROWS, 80), jnp.float32),
        mesh=mesh,
        scratch_types=[
            pltpu.VMEM((2, _CH), jnp.int32),
            pltpu.VMEM((2, _CH), jnp.int32),
            pltpu.VMEM((_CH, 64), jnp.float32),
            pltpu.VMEM((_CH, 64), jnp.float32),
            pltpu.VMEM((_CH, 16), jnp.float32),
            pltpu.VMEM((_CH, 16), jnp.float32),
            pltpu.VMEM((_CH, 16), jnp.float32),
            pltpu.VMEM((_CH, 16), jnp.float32),
            pltpu.VMEM((_CH, 80), jnp.float32),
            pltpu.VMEM((_CH, 80), jnp.float32),
            pltpu.VMEM((4, 16), jnp.float32),
            pltpu.SemaphoreType.DMA,
            pltpu.SemaphoreType.DMA,
            pltpu.SemaphoreType.DMA,
            pltpu.SemaphoreType.DMA,
            pltpu.SemaphoreType.DMA,
            pltpu.SemaphoreType.DMA,
            pltpu.SemaphoreType.DMA,
            pltpu.SemaphoreType.DMA,
            pltpu.VMEM_SHARED((_AROWS, 80), jnp.float32),
        ],
        compiler_params=pltpu.CompilerParams(
            use_tc_tiling_on_sc=False, needs_layout_passes=False),
    )
    return f(sd, h1tab, asad1, adas1)


def _run_sc2(sd, tab2):
    mesh = plsc.VectorSubcoreMesh(core_axis_name="c", subcore_axis_name="s")
    f = pl.kernel(
        _sc_layer2,
        out_type=jax.ShapeDtypeStruct((2, _AROWS, 16), jnp.float32),
        mesh=mesh,
        scratch_types=[
            pltpu.VMEM((2, _CH), jnp.int32),
            pltpu.VMEM((2, _CH), jnp.int32),
            pltpu.VMEM((_CH, 16), jnp.float32),
            pltpu.VMEM((_CH, 16), jnp.float32),
            pltpu.VMEM((_CH, 16), jnp.float32),
            pltpu.VMEM((_CH, 16), jnp.float32),
            pltpu.VMEM((_CH, 16), jnp.float32),
            pltpu.VMEM((_CH, 16), jnp.float32),
            pltpu.SemaphoreType.DMA,
            pltpu.SemaphoreType.DMA,
            pltpu.SemaphoreType.DMA,
            pltpu.SemaphoreType.DMA,
            pltpu.SemaphoreType.DMA,
            pltpu.SemaphoreType.DMA,
            pltpu.VMEM_SHARED((_AROWS, 16), jnp.float32),
        ],
        compiler_params=pltpu.CompilerParams(
            use_tc_tiling_on_sc=False, needs_layout_passes=False),
    )
    return f(sd, tab2)


def kernel(x, edge_index, W1, a_src1, a_dst1, b1, W2, a_src2, a_dst2, b2):
    loops = jnp.arange(_N, dtype=jnp.int32)
    src = jnp.concatenate([edge_index[0], loops])
    dst = jnp.concatenate([edge_index[1], loops])
    npad = _EPAD - src.shape[0]
    srcp = jnp.concatenate([src, jnp.zeros((npad,), jnp.int32)])
    dstp = jnp.concatenate([dst, jnp.full((npad,), _N, jnp.int32)])
    sd = jnp.stack([srcp.reshape(-1, _CH), dstp.reshape(-1, _CH)], axis=1)

    eye8 = jnp.eye(8, dtype=jnp.float32)
    a1s = (eye8[:, None, :] * a_src1[:, :, None]).reshape(64, 8)
    a1d = (eye8[:, None, :] * a_dst1[:, :, None]).reshape(64, 8)
    A1 = jnp.concatenate([a1s, a1d], axis=1)
    A1r = jnp.concatenate([a1d, a1s], axis=1)

    E64 = jnp.concatenate(
        [jnp.eye(64, dtype=jnp.float32), jnp.zeros((16, 64), jnp.float32)], 0)
    BDEN = jnp.concatenate(
        [jnp.zeros((64, 64), jnp.float32),
         jnp.repeat(eye8, 8, axis=1),
         jnp.zeros((8, 64), jnp.float32)], 0)

    M = jnp.zeros((7, 16), jnp.float32)
    M = M.at[:, 0:7].set(jnp.eye(7, dtype=jnp.float32))
    M = M.at[:, 7].set(a_src2[0])
    M = M.at[:, 8].set(a_dst2[0])
    W2M = W2 @ M

    D16 = jnp.zeros((16, 16), jnp.float32).at[7, :].set(1.0)
    b2p = jnp.concatenate([b2, jnp.zeros((9,), jnp.float32)])

    h1tab, asad1, adas1 = pl.pallas_call(
        _tc_prep1,
        out_shape=(jax.ShapeDtypeStruct((_N, 64), jnp.float32),
                   jax.ShapeDtypeStruct((_N, 16), jnp.float32),
                   jax.ShapeDtypeStruct((_N, 16), jnp.float32)),
    )(x, W1, A1, A1r)

    part1 = _run_sc1(sd, h1tab, asad1, adas1)

    tab2 = pl.pallas_call(
        _tc_mid,
        out_shape=jax.ShapeDtypeStruct((_N, 16), jnp.float32),
    )(part1, E64, BDEN, b1, W2M)

    part2 = _run_sc2(sd, tab2)

    out16 = pl.pallas_call(
        _tc_out,
        out_shape=jax.ShapeDtypeStruct((_N, 16), jnp.float32),
    )(part2, D16, b2p)

    return out16[:, :7]


# speedup vs baseline: 141.9519x; 2.0846x over previous
import functools

import jax
import jax.numpy as jnp
from jax import lax
from jax.experimental import pallas as pl
from jax.experimental.pallas import tpu as pltpu
from jax.experimental.pallas import tpu_sc as plsc

_N = 10000
_CH = 128
_CHUNKS = 81
_NW = 32
_EPAD = _NW * _CHUNKS * _CH
_AROWS = 10240
_ZROWS = _AROWS // 16


def _tc_prep1(x_ref, w1_ref, a1_ref, a1r_ref, h_ref, asad_ref, adas_ref):
    h = jnp.dot(x_ref[...], w1_ref[...], preferred_element_type=jnp.float32)
    h_ref[...] = h
    asad_ref[...] = jnp.dot(h, a1_ref[...], preferred_element_type=jnp.float32)
    adas_ref[...] = jnp.dot(h, a1r_ref[...], preferred_element_type=jnp.float32)


def _tc_mid(p_ref, e64_ref, bden_ref, b1_ref, w2m_ref, tab_ref):
    pp = p_ref[0] + p_ref[1]
    pp = pp[:_N]
    msg = jnp.dot(pp, e64_ref[...], preferred_element_type=jnp.float32)
    den = jnp.dot(pp, bden_ref[...], preferred_element_type=jnp.float32)
    x1 = msg / den + b1_ref[...]
    x1 = jnp.where(x1 > 0, x1, jnp.exp(jnp.minimum(x1, 0.0)) - 1.0)
    tab_ref[...] = jnp.dot(x1, w2m_ref[...], preferred_element_type=jnp.float32)


def _tc_out(p_ref, d16_ref, b2p_ref, o_ref):
    pp = p_ref[0, :_N] + p_ref[1, :_N]
    den = jnp.dot(pp, d16_ref[...], preferred_element_type=jnp.float32)
    o = pp / den + b2p_ref[...]
    mask = lax.broadcasted_iota(jnp.int32, (1, 16), 1) < 7
    om = jnp.where(mask, o, -1e30)
    m = jnp.max(om, axis=1, keepdims=True)
    ez = jnp.where(mask, jnp.exp(om - m), 0.0)
    lse = jnp.log(jnp.sum(ez, axis=1, keepdims=True)) + m
    o_ref[...] = o - lse


def _fill_i32(ref_row, iota, val):
    for k in range(8):
        ref_row[pl.ds(k * 16, 16)] = iota * 0 + val


def _sc_layer1(sd_hbm, htab_hbm, asad_hbm, adas_hbm, out_hbm,
               idx0, idx1, h0, h1, s0, s1, d0, d1, m0, m1, ex_s,
               sh0, sh1, ss0, ss1, sd0, sd1, sc0, sc1, accum):
    c = lax.axis_index("c")
    s = lax.axis_index("s")
    w = c * 16 + s
    iota = lax.iota(jnp.int32, 16)
    lane_lt8 = iota < 8
    zeros16 = jnp.zeros((16,), jnp.float32)
    B0 = (idx0, h0, s0, d0, m0, sh0, ss0, sd0, sc0)
    B1 = (idx1, h1, s1, d1, m1, sh1, ss1, sd1, sc1)

    def _zrow(i, _):
        for v in range(5):
            m0[i, pl.ds(v * 16, 16)] = zeros16
            m1[i, pl.ds(v * 16, 16)] = zeros16
        return 0
    lax.fori_loop(0, _CH, _zrow, 0)
    for k in range(_ZROWS // _CH):
        pltpu.sync_copy(m0, accum.at[pl.ds(s * _ZROWS + k * _CH, _CH)])
    plsc.subcore_barrier()

    def scatter_issue(B):
        idx, hb, sb, db, mb, sh, ss, sd, sc = B
        pltpu.async_copy(mb, accum.at[idx.at[1]], sc, add=True)

    def scatter_wait(B):
        idx, hb, sb, db, mb, sh, ss, sd, sc = B
        pltpu.make_async_copy(mb, accum.at[idx.at[1]], sc).wait()

    def load_issue(chunk, B):
        idx, hb, sb, db, mb, sh, ss, sd, sc = B
        scatter_wait(B)
        pltpu.sync_copy(sd_hbm.at[chunk], idx)
        pltpu.async_copy(htab_hbm.at[idx.at[0]], hb, sh)
        pltpu.async_copy(asad_hbm.at[idx.at[0]], sb, ss)
        pltpu.async_copy(adas_hbm.at[idx.at[1]], db, sd)

    def process(B):
        idx, hb, sb, db, mb, sh, ss, sd, sc = B
        pltpu.make_async_copy(asad_hbm.at[idx.at[0]], sb, ss).wait()
        pltpu.make_async_copy(adas_hbm.at[idx.at[1]], db, sd).wait()
        pltpu.make_async_copy(htab_hbm.at[idx.at[0]], hb, sh).wait()

        @plsc.parallel_loop(0, _CH, 1, unroll=4)
        def _edge(i):
            e = sb[i, :] + db[i, :]
            e = jnp.where(e > 0, e, 0.2 * e)
            ex = jnp.exp(e)
            ex_s[i, :] = ex
            mb[i, pl.ds(64, 16)] = jnp.where(lane_lt8, ex, 0.0)
            bi = iota * 0 + i
            for v in range(4):
                av = plsc.load_gather(ex_s, [bi, 2 * v + iota // 8])
                mb[i, pl.ds(v * 16, 16)] = hb[i, pl.ds(v * 16, 16)] * av
        scatter_issue(B)

    for B in (B0, B1):
        _fill_i32(B[0].at[1], iota, _N)
        scatter_issue(B)

    base = w * _CHUNKS
    load_issue(base, B0)

    def _pair(t, _):
        j0 = base + 2 * t
        load_issue(j0 + 1, B1)
        process(B0)
        load_issue(j0 + 2, B0)
        process(B1)
        return 0
    lax.fori_loop(0, (_CHUNKS - 1) // 2, _pair, 0)
    process(B0)
    scatter_wait(B0)
    scatter_wait(B1)

    plsc.subcore_barrier()
    for k in range(_ZROWS // _CH):
        r0 = s * _ZROWS + k * _CH
        pltpu.sync_copy(accum.at[pl.ds(r0, _CH)], out_hbm.at[c, pl.ds(r0, _CH)])


def _sc_layer2(sd_hbm, tab_hbm, out_hbm,
               idx0, idx1, s0, s1, d0, d1, m0, m1, ex_s,
               ss0, ss1, sd0, sd1, sc0, sc1, accum):
    c = lax.axis_index("c")
    s = lax.axis_index("s")
    w = c * 16 + s
    iota = lax.iota(jnp.int32, 16)
    zeros16 = jnp.zeros((16,), jnp.float32)
    B0 = (idx0, s0, d0, m0, ss0, sd0, sc0)
    B1 = (idx1, s1, d1, m1, ss1, sd1, sc1)

    def _zrow(i, _):
        m0[i, :] = zeros16
        m1[i, :] = zeros16
        return 0
    lax.fori_loop(0, _CH, _zrow, 0)
    for k in range(_ZROWS // _CH):
        pltpu.sync_copy(m0, accum.at[pl.ds(s * _ZROWS + k * _CH, _CH)])
    plsc.subcore_barrier()

    def scatter_issue(B):
        idx, sb, db, mb, ss, sd, sc = B
        pltpu.async_copy(mb, accum.at[idx.at[1]], sc, add=True)

    def scatter_wait(B):
        idx, sb, db, mb, ss, sd, sc = B
        pltpu.make_async_copy(mb, accum.at[idx.at[1]], sc).wait()

    def load_issue(chunk, B):
        idx, sb, db, mb, ss, sd, sc = B
        scatter_wait(B)
        pltpu.sync_copy(sd_hbm.at[chunk], idx)
        pltpu.async_copy(tab_hbm.at[idx.at[0]], sb, ss)
        pltpu.async_copy(tab_hbm.at[idx.at[1]], db, sd)

    def process(B):
        idx, sb, db, mb, ss, sd, sc = B
        pltpu.make_async_copy(tab_hbm.at[idx.at[0]], sb, ss).wait()
        pltpu.make_async_copy(tab_hbm.at[idx.at[1]], db, sd).wait()

        @plsc.parallel_loop(0, _CH // 16, 1, unroll=2)
        def _grp(g):
            rows = g * 16 + iota
            e = (plsc.load_gather(sb, [rows, iota * 0 + 7])
                 + plsc.load_gather(db, [rows, iota * 0 + 8]))
            e = jnp.where(e > 0, e, 0.2 * e)
            ex_s[g, :] = jnp.exp(e)
            bg = iota * 0 + g
            for u in range(16):
                i = g * 16 + u
                srow = sb[i, :]
                exu = plsc.load_gather(ex_s, [bg, iota * 0 + u])
                t = jnp.where(iota == 7, 1.0, srow)
                mb[i, :] = jnp.where(iota < 8, t * exu, 0.0)
        scatter_issue(B)

    for B in (B0, B1):
        _fill_i32(B[0].at[1], iota, _N)
        scatter_issue(B)

    base = w * _CHUNKS
    load_issue(base, B0)

    def _pair(t, _):
        j0 = base + 2 * t
        load_issue(j0 + 1, B1)
        process(B0)
        load_issue(j0 + 2, B0)
        process(B1)
        return 0
    lax.fori_loop(0, (_CHUNKS - 1) // 2, _pair, 0)
    process(B0)
    scatter_wait(B0)
    scatter_wait(B1)

    plsc.subcore_barrier()
    for k in range(_ZROWS // _CH):
        r0 = s * _ZROWS + k * _CH
        pltpu.sync_copy(accum.at[pl.ds(r0, _CH)], out_hbm.at[c, pl.ds(r0, _CH)])


def _run_sc1(sd, h1tab, asad1, adas1):
    mesh = plsc.VectorSubcoreMesh(core_axis_name="c", subcore_axis_name="s")
    f = pl.kernel(
        _sc_layer1,
        out_type=jax.ShapeDtypeStruct((2, _AROWS, 80), jnp.float32),
        mesh=mesh,
        scratch_types=[
            pltpu.VMEM((2, _CH), jnp.int32),
            pltpu.VMEM((2, _CH), jnp.int32),
            pltpu.VMEM((_CH, 64), jnp.float32),
            pltpu.VMEM((_CH, 64), jnp.float32),
            pltpu.VMEM((_CH, 16), jnp.float32),
            pltpu.VMEM((_CH, 16), jnp.float32),
            pltpu.VMEM((_CH, 16), jnp.float32),
            pltpu.VMEM((_CH, 16), jnp.float32),
            pltpu.VMEM((_CH, 80), jnp.float32),
            pltpu.VMEM((_CH, 80), jnp.float32),
            pltpu.VMEM((_CH, 16), jnp.float32),
            pltpu.SemaphoreType.DMA,
            pltpu.SemaphoreType.DMA,
            pltpu.SemaphoreType.DMA,
            pltpu.SemaphoreType.DMA,
            pltpu.SemaphoreType.DMA,
            pltpu.SemaphoreType.DMA,
            pltpu.SemaphoreType.DMA,
            pltpu.SemaphoreType.DMA,
            pltpu.VMEM_SHARED((_AROWS, 80), jnp.float32),
        ],
        compiler_params=pltpu.CompilerParams(
            use_tc_tiling_on_sc=False, needs_layout_passes=False),
    )
    return f(sd, h1tab, asad1, adas1)


def _run_sc2(sd, tab2):
    mesh = plsc.VectorSubcoreMesh(core_axis_name="c", subcore_axis_name="s")
    f = pl.kernel(
        _sc_layer2,
        out_type=jax.ShapeDtypeStruct((2, _AROWS, 16), jnp.float32),
        mesh=mesh,
        scratch_types=[
            pltpu.VMEM((2, _CH), jnp.int32),
            pltpu.VMEM((2, _CH), jnp.int32),
            pltpu.VMEM((_CH, 16), jnp.float32),
            pltpu.VMEM((_CH, 16), jnp.float32),
            pltpu.VMEM((_CH, 16), jnp.float32),
            pltpu.VMEM((_CH, 16), jnp.float32),
            pltpu.VMEM((_CH, 16), jnp.float32),
            pltpu.VMEM((_CH, 16), jnp.float32),
            pltpu.VMEM((_CH // 16, 16), jnp.float32),
            pltpu.SemaphoreType.DMA,
            pltpu.SemaphoreType.DMA,
            pltpu.SemaphoreType.DMA,
            pltpu.SemaphoreType.DMA,
            pltpu.SemaphoreType.DMA,
            pltpu.SemaphoreType.DMA,
            pltpu.VMEM_SHARED((_AROWS, 16), jnp.float32),
        ],
        compiler_params=pltpu.CompilerParams(
            use_tc_tiling_on_sc=False, needs_layout_passes=False),
    )
    return f(sd, tab2)


def kernel(x, edge_index, W1, a_src1, a_dst1, b1, W2, a_src2, a_dst2, b2):
    loops = jnp.arange(_N, dtype=jnp.int32)
    src = jnp.concatenate([edge_index[0], loops])
    dst = jnp.concatenate([edge_index[1], loops])
    npad = _EPAD - src.shape[0]
    srcp = jnp.concatenate([src, jnp.zeros((npad,), jnp.int32)])
    dstp = jnp.concatenate([dst, jnp.full((npad,), _N, jnp.int32)])
    sd = jnp.stack([srcp.reshape(-1, _CH), dstp.reshape(-1, _CH)], axis=1)

    eye8 = jnp.eye(8, dtype=jnp.float32)
    a1s = (eye8[:, None, :] * a_src1[:, :, None]).reshape(64, 8)
    a1d = (eye8[:, None, :] * a_dst1[:, :, None]).reshape(64, 8)
    A1 = jnp.concatenate([a1s, a1d], axis=1)
    A1r = jnp.concatenate([a1d, a1s], axis=1)

    E64 = jnp.concatenate(
        [jnp.eye(64, dtype=jnp.float32), jnp.zeros((16, 64), jnp.float32)], 0)
    BDEN = jnp.concatenate(
        [jnp.zeros((64, 64), jnp.float32),
         jnp.repeat(eye8, 8, axis=1),
         jnp.zeros((8, 64), jnp.float32)], 0)

    M = jnp.zeros((7, 16), jnp.float32)
    M = M.at[:, 0:7].set(jnp.eye(7, dtype=jnp.float32))
    M = M.at[:, 7].set(a_src2[0])
    M = M.at[:, 8].set(a_dst2[0])
    W2M = W2 @ M

    D16 = jnp.zeros((16, 16), jnp.float32).at[7, :].set(1.0)
    b2p = jnp.concatenate([b2, jnp.zeros((9,), jnp.float32)])

    h1tab, asad1, adas1 = pl.pallas_call(
        _tc_prep1,
        out_shape=(jax.ShapeDtypeStruct((_N, 64), jnp.float32),
                   jax.ShapeDtypeStruct((_N, 16), jnp.float32),
                   jax.ShapeDtypeStruct((_N, 16), jnp.float32)),
    )(x, W1, A1, A1r)

    part1 = _run_sc1(sd, h1tab, asad1, adas1)

    tab2 = pl.pallas_call(
        _tc_mid,
        out_shape=jax.ShapeDtypeStruct((_N, 16), jnp.float32),
    )(part1, E64, BDEN, b1, W2M)

    part2 = _run_sc2(sd, tab2)

    out16 = pl.pallas_call(
        _tc_out,
        out_shape=jax.ShapeDtypeStruct((_N, 16), jnp.float32),
    )(part2, D16, b2p)

    return out16[:, :7]


# speedup vs baseline: 159.4605x; 1.1233x over previous
import functools

import jax
import jax.numpy as jnp
from jax import lax
from jax.experimental import pallas as pl
from jax.experimental.pallas import tpu as pltpu
from jax.experimental.pallas import tpu_sc as plsc

_N = 10000
_CH = 128
_CHUNKS = 81
_NW = 32
_EPAD = _NW * _CHUNKS * _CH
_AROWS = 10240
_ZROWS = _AROWS // 16


def _tc_prep1(x_ref, w1_ref, a1_ref, a1r_ref, h_ref, asad_ref, adas_ref):
    h = jnp.dot(x_ref[...], w1_ref[...], preferred_element_type=jnp.float32)
    h_ref[...] = h.astype(jnp.bfloat16)
    asad_ref[...] = jnp.dot(h, a1_ref[...], preferred_element_type=jnp.float32)
    adas_ref[...] = jnp.dot(h, a1r_ref[...], preferred_element_type=jnp.float32)


def _tc_mid(p_ref, e64_ref, bden_ref, b1_ref, w2m_ref, tab_ref):
    pp = p_ref[0] + p_ref[1]
    pp = pp[:_N]
    msg = jnp.dot(pp, e64_ref[...], preferred_element_type=jnp.float32)
    den = jnp.dot(pp, bden_ref[...], preferred_element_type=jnp.float32)
    x1 = msg / den + b1_ref[...]
    x1 = jnp.where(x1 > 0, x1, jnp.exp(jnp.minimum(x1, 0.0)) - 1.0)
    tab_ref[...] = jnp.dot(x1, w2m_ref[...], preferred_element_type=jnp.float32)


def _tc_out(p_ref, d16_ref, b2p_ref, o_ref):
    pp = p_ref[0, :_N] + p_ref[1, :_N]
    den = jnp.dot(pp, d16_ref[...], preferred_element_type=jnp.float32)
    o = pp / den + b2p_ref[...]
    mask = lax.broadcasted_iota(jnp.int32, (1, 16), 1) < 7
    om = jnp.where(mask, o, -1e30)
    m = jnp.max(om, axis=1, keepdims=True)
    ez = jnp.where(mask, jnp.exp(om - m), 0.0)
    lse = jnp.log(jnp.sum(ez, axis=1, keepdims=True)) + m
    o_ref[...] = o - lse


def _fill_i32(ref_row, iota, val):
    for k in range(8):
        ref_row[pl.ds(k * 16, 16)] = iota * 0 + val


def _sc_layer1(sd_hbm, htab_hbm, asad_hbm, adas_hbm, out_hbm,
               idx0, idx1, h0, h1, s0, s1, d0, d1, m0, m1, ex_s,
               sh0, sh1, ss0, ss1, sd0, sd1, sc0, sc1, accum):
    c = lax.axis_index("c")
    s = lax.axis_index("s")
    w = c * 16 + s
    iota = lax.iota(jnp.int32, 16)
    lane_lt8 = iota < 8
    zeros16 = jnp.zeros((16,), jnp.float32)
    B0 = (idx0, h0, s0, d0, m0, sh0, ss0, sd0, sc0)
    B1 = (idx1, h1, s1, d1, m1, sh1, ss1, sd1, sc1)

    def _zrow(i, _):
        for v in range(5):
            m0[i, pl.ds(v * 16, 16)] = zeros16
            m1[i, pl.ds(v * 16, 16)] = zeros16
        return 0
    lax.fori_loop(0, _CH, _zrow, 0)
    for k in range(_ZROWS // _CH):
        pltpu.sync_copy(m0, accum.at[pl.ds(s * _ZROWS + k * _CH, _CH)])
    plsc.subcore_barrier()

    def scatter_issue(B):
        idx, hb, sb, db, mb, sh, ss, sd, sc = B
        pltpu.async_copy(mb, accum.at[idx.at[1]], sc, add=True)

    def scatter_wait(B):
        idx, hb, sb, db, mb, sh, ss, sd, sc = B
        pltpu.make_async_copy(mb, accum.at[idx.at[1]], sc).wait()

    def load_issue(chunk, B):
        idx, hb, sb, db, mb, sh, ss, sd, sc = B
        scatter_wait(B)
        pltpu.sync_copy(sd_hbm.at[chunk], idx)
        pltpu.async_copy(htab_hbm.at[idx.at[0]], hb, sh)
        pltpu.async_copy(asad_hbm.at[idx.at[0]], sb, ss)
        pltpu.async_copy(adas_hbm.at[idx.at[1]], db, sd)

    def process(B):
        idx, hb, sb, db, mb, sh, ss, sd, sc = B
        pltpu.make_async_copy(asad_hbm.at[idx.at[0]], sb, ss).wait()
        pltpu.make_async_copy(adas_hbm.at[idx.at[1]], db, sd).wait()
        pltpu.make_async_copy(htab_hbm.at[idx.at[0]], hb, sh).wait()

        @plsc.parallel_loop(0, _CH, 1, unroll=4)
        def _edge(i):
            e = sb[i, :] + db[i, :]
            e = jnp.where(e > 0, e, 0.2 * e)
            ex = jnp.exp(e)
            ex_s[i, :] = ex
            mb[i, pl.ds(64, 16)] = jnp.where(lane_lt8, ex, 0.0)
            bi = iota * 0 + i
            for v in range(2):
                y = hb[i, pl.ds(32 * v, 32)]
                a, b = plsc.unpack(y, format=plsc.PackFormat.INTERLEAVED)
                av = plsc.load_gather(ex_s, [bi, 4 * v + iota // 4])
                mb[i, pl.ds(32 * v, 16)] = a * av
                mb[i, pl.ds(32 * v + 16, 16)] = b * av
        scatter_issue(B)

    for B in (B0, B1):
        _fill_i32(B[0].at[1], iota, _N)
        scatter_issue(B)

    base = w * _CHUNKS
    load_issue(base, B0)

    def _pair(t, _):
        j0 = base + 2 * t
        load_issue(j0 + 1, B1)
        process(B0)
        load_issue(j0 + 2, B0)
        process(B1)
        return 0
    lax.fori_loop(0, (_CHUNKS - 1) // 2, _pair, 0)
    process(B0)
    scatter_wait(B0)
    scatter_wait(B1)

    plsc.subcore_barrier()
    for k in range(_ZROWS // _CH):
        r0 = s * _ZROWS + k * _CH
        pltpu.sync_copy(accum.at[pl.ds(r0, _CH)], out_hbm.at[c, pl.ds(r0, _CH)])


def _sc_layer2(sd_hbm, tab_hbm, out_hbm,
               idx0, idx1, s0, s1, d0, d1, m0, m1, ex_s,
               ss0, ss1, sd0, sd1, sc0, sc1, accum):
    c = lax.axis_index("c")
    s = lax.axis_index("s")
    w = c * 16 + s
    iota = lax.iota(jnp.int32, 16)
    zeros16 = jnp.zeros((16,), jnp.float32)
    B0 = (idx0, s0, d0, m0, ss0, sd0, sc0)
    B1 = (idx1, s1, d1, m1, ss1, sd1, sc1)

    def _zrow(i, _):
        m0[i, :] = zeros16
        m1[i, :] = zeros16
        return 0
    lax.fori_loop(0, _CH, _zrow, 0)
    for k in range(_ZROWS // _CH):
        pltpu.sync_copy(m0, accum.at[pl.ds(s * _ZROWS + k * _CH, _CH)])
    plsc.subcore_barrier()

    def scatter_issue(B):
        idx, sb, db, mb, ss, sd, sc = B
        pltpu.async_copy(mb, accum.at[idx.at[1]], sc, add=True)

    def scatter_wait(B):
        idx, sb, db, mb, ss, sd, sc = B
        pltpu.make_async_copy(mb, accum.at[idx.at[1]], sc).wait()

    def load_issue(chunk, B):
        idx, sb, db, mb, ss, sd, sc = B
        scatter_wait(B)
        pltpu.sync_copy(sd_hbm.at[chunk], idx)
        pltpu.async_copy(tab_hbm.at[idx.at[0]], sb, ss)
        pltpu.async_copy(tab_hbm.at[idx.at[1]], db, sd)

    def process(B):
        idx, sb, db, mb, ss, sd, sc = B
        pltpu.make_async_copy(tab_hbm.at[idx.at[0]], sb, ss).wait()
        pltpu.make_async_copy(tab_hbm.at[idx.at[1]], db, sd).wait()

        @plsc.parallel_loop(0, _CH // 16, 1, unroll=2)
        def _grp(g):
            rows = g * 16 + iota
            e = (plsc.load_gather(sb, [rows, iota * 0 + 7])
                 + plsc.load_gather(db, [rows, iota * 0 + 8]))
            e = jnp.where(e > 0, e, 0.2 * e)
            ex_s[g, :] = jnp.exp(e)
            bg = iota * 0 + g
            for u in range(16):
                i = g * 16 + u
                srow = sb[i, :]
                exu = plsc.load_gather(ex_s, [bg, iota * 0 + u])
                t = jnp.where(iota == 7, 1.0, srow)
                mb[i, :] = jnp.where(iota < 8, t * exu, 0.0)
        scatter_issue(B)

    for B in (B0, B1):
        _fill_i32(B[0].at[1], iota, _N)
        scatter_issue(B)

    base = w * _CHUNKS
    load_issue(base, B0)

    def _pair(t, _):
        j0 = base + 2 * t
        load_issue(j0 + 1, B1)
        process(B0)
        load_issue(j0 + 2, B0)
        process(B1)
        return 0
    lax.fori_loop(0, (_CHUNKS - 1) // 2, _pair, 0)
    process(B0)
    scatter_wait(B0)
    scatter_wait(B1)

    plsc.subcore_barrier()
    for k in range(_ZROWS // _CH):
        r0 = s * _ZROWS + k * _CH
        pltpu.sync_copy(accum.at[pl.ds(r0, _CH)], out_hbm.at[c, pl.ds(r0, _CH)])


def _run_sc1(sd, h1tab, asad1, adas1):
    mesh = plsc.VectorSubcoreMesh(core_axis_name="c", subcore_axis_name="s")
    f = pl.kernel(
        _sc_layer1,
        out_type=jax.ShapeDtypeStruct((2, _AROWS, 80), jnp.float32),
        mesh=mesh,
        scratch_types=[
            pltpu.VMEM((2, _CH), jnp.int32),
            pltpu.VMEM((2, _CH), jnp.int32),
            pltpu.VMEM((_CH, 64), jnp.bfloat16),
            pltpu.VMEM((_CH, 64), jnp.bfloat16),
            pltpu.VMEM((_CH, 16), jnp.float32),
            pltpu.VMEM((_CH, 16), jnp.float32),
            pltpu.VMEM((_CH, 16), jnp.float32),
            pltpu.VMEM((_CH, 16), jnp.float32),
            pltpu.VMEM((_CH, 80), jnp.float32),
            pltpu.VMEM((_CH, 80), jnp.float32),
            pltpu.VMEM((_CH, 16), jnp.float32),
            pltpu.SemaphoreType.DMA,
            pltpu.SemaphoreType.DMA,
            pltpu.SemaphoreType.DMA,
            pltpu.SemaphoreType.DMA,
            pltpu.SemaphoreType.DMA,
            pltpu.SemaphoreType.DMA,
            pltpu.SemaphoreType.DMA,
            pltpu.SemaphoreType.DMA,
            pltpu.VMEM_SHARED((_AROWS, 80), jnp.float32),
        ],
        compiler_params=pltpu.CompilerParams(
            use_tc_tiling_on_sc=False, needs_layout_passes=False),
    )
    return f(sd, h1tab, asad1, adas1)


def _run_sc2(sd, tab2):
    mesh = plsc.VectorSubcoreMesh(core_axis_name="c", subcore_axis_name="s")
    f = pl.kernel(
        _sc_layer2,
        out_type=jax.ShapeDtypeStruct((2, _AROWS, 16), jnp.float32),
        mesh=mesh,
        scratch_types=[
            pltpu.VMEM((2, _CH), jnp.int32),
            pltpu.VMEM((2, _CH), jnp.int32),
            pltpu.VMEM((_CH, 16), jnp.float32),
            pltpu.VMEM((_CH, 16), jnp.float32),
            pltpu.VMEM((_CH, 16), jnp.float32),
            pltpu.VMEM((_CH, 16), jnp.float32),
            pltpu.VMEM((_CH, 16), jnp.float32),
            pltpu.VMEM((_CH, 16), jnp.float32),
            pltpu.VMEM((_CH // 16, 16), jnp.float32),
            pltpu.SemaphoreType.DMA,
            pltpu.SemaphoreType.DMA,
            pltpu.SemaphoreType.DMA,
            pltpu.SemaphoreType.DMA,
            pltpu.SemaphoreType.DMA,
            pltpu.SemaphoreType.DMA,
            pltpu.VMEM_SHARED((_AROWS, 16), jnp.float32),
        ],
        compiler_params=pltpu.CompilerParams(
            use_tc_tiling_on_sc=False, needs_layout_passes=False),
    )
    return f(sd, tab2)


def kernel(x, edge_index, W1, a_src1, a_dst1, b1, W2, a_src2, a_dst2, b2):
    loops = jnp.arange(_N, dtype=jnp.int32)
    src = jnp.concatenate([edge_index[0], loops])
    dst = jnp.concatenate([edge_index[1], loops])
    npad = _EPAD - src.shape[0]
    srcp = jnp.concatenate([src, jnp.zeros((npad,), jnp.int32)])
    dstp = jnp.concatenate([dst, jnp.full((npad,), _N, jnp.int32)])
    sd = jnp.stack([srcp.reshape(-1, _CH), dstp.reshape(-1, _CH)], axis=1)

    eye8 = jnp.eye(8, dtype=jnp.float32)
    a1s = (eye8[:, None, :] * a_src1[:, :, None]).reshape(64, 8)
    a1d = (eye8[:, None, :] * a_dst1[:, :, None]).reshape(64, 8)
    A1 = jnp.concatenate([a1s, a1d], axis=1)
    A1r = jnp.concatenate([a1d, a1s], axis=1)

    perm = jnp.asarray([(j // 32) * 32 + 2 * (j % 16) + ((j // 16) % 2)
                        for j in range(64)], dtype=jnp.int32)
    E64 = jnp.zeros((80, 64), jnp.float32).at[jnp.arange(64), perm].set(1.0)
    BDEN = jnp.concatenate(
        [jnp.zeros((64, 64), jnp.float32),
         jnp.repeat(eye8, 8, axis=1),
         jnp.zeros((8, 64), jnp.float32)], 0)

    M = jnp.zeros((7, 16), jnp.float32)
    M = M.at[:, 0:7].set(jnp.eye(7, dtype=jnp.float32))
    M = M.at[:, 7].set(a_src2[0])
    M = M.at[:, 8].set(a_dst2[0])
    W2M = W2 @ M

    D16 = jnp.zeros((16, 16), jnp.float32).at[7, :].set(1.0)
    b2p = jnp.concatenate([b2, jnp.zeros((9,), jnp.float32)])

    h1tab, asad1, adas1 = pl.pallas_call(
        _tc_prep1,
        out_shape=(jax.ShapeDtypeStruct((_N, 64), jnp.bfloat16),
                   jax.ShapeDtypeStruct((_N, 16), jnp.float32),
                   jax.ShapeDtypeStruct((_N, 16), jnp.float32)),
    )(x, W1, A1, A1r)

    part1 = _run_sc1(sd, h1tab, asad1, adas1)

    tab2 = pl.pallas_call(
        _tc_mid,
        out_shape=jax.ShapeDtypeStruct((_N, 16), jnp.float32),
    )(part1, E64, BDEN, b1, W2M)

    part2 = _run_sc2(sd, tab2)

    out16 = pl.pallas_call(
        _tc_out,
        out_shape=jax.ShapeDtypeStruct((_N, 16), jnp.float32),
    )(part2, D16, b2p)

    return out16[:, :7]


# speedup vs baseline: 194.7629x; 1.2214x over previous
import functools

import jax
import jax.numpy as jnp
from jax import lax
from jax.experimental import pallas as pl
from jax.experimental.pallas import tpu as pltpu
from jax.experimental.pallas import tpu_sc as plsc

_N = 10000
_CH = 128
_CHUNKS = 81
_NW = 32
_EPAD = _NW * _CHUNKS * _CH
_AROWS = 10240
_ZROWS = _AROWS // 16


def _tc_prep1(x_ref, w1_ref, a1_ref, a1r_ref, h_ref, asad_ref, adas_ref):
    h = jnp.dot(x_ref[...], w1_ref[...], preferred_element_type=jnp.float32)
    h_ref[...] = h.astype(jnp.bfloat16)
    asad_ref[...] = jnp.dot(h, a1_ref[...], preferred_element_type=jnp.float32)
    adas_ref[...] = jnp.dot(h, a1r_ref[...], preferred_element_type=jnp.float32)


def _tc_mid(p_ref, e64_ref, bden_ref, b1_ref, w2m_ref, tab_ref):
    pp = p_ref[0] + p_ref[1]
    pp = pp[:_N]
    msg = jnp.dot(pp, e64_ref[...], preferred_element_type=jnp.float32)
    den = jnp.dot(pp, bden_ref[...], preferred_element_type=jnp.float32)
    x1 = msg / den + b1_ref[...]
    x1 = jnp.where(x1 > 0, x1, jnp.exp(jnp.minimum(x1, 0.0)) - 1.0)
    tab_ref[...] = jnp.dot(x1, w2m_ref[...], preferred_element_type=jnp.float32)


def _tc_out(p_ref, d16_ref, b2p_ref, o_ref):
    pp = p_ref[0, :_N] + p_ref[1, :_N]
    den = jnp.dot(pp, d16_ref[...], preferred_element_type=jnp.float32)
    o = pp / den + b2p_ref[...]
    mask = lax.broadcasted_iota(jnp.int32, (1, 16), 1) < 7
    om = jnp.where(mask, o, -1e30)
    m = jnp.max(om, axis=1, keepdims=True)
    ez = jnp.where(mask, jnp.exp(om - m), 0.0)
    lse = jnp.log(jnp.sum(ez, axis=1, keepdims=True)) + m
    o_ref[...] = o - lse


def _fill_i32(ref_row, iota, val):
    for k in range(8):
        ref_row[pl.ds(k * 16, 16)] = iota * 0 + val


def _sc_layer1(sd_hbm, htab_hbm, asad_hbm, adas_hbm, out_hbm,
               sd_l, idxp, h0, h1, s0, s1, d0, d1, m0, m1, ex_s,
               sh0, sh1, ss0, ss1, sd0, sd1, sc0, sc1, accum):
    c = lax.axis_index("c")
    s = lax.axis_index("s")
    w = c * 16 + s
    iota = lax.iota(jnp.int32, 16)
    lane_lt8 = iota < 8
    zeros16 = jnp.zeros((16,), jnp.float32)
    B0 = (h0, s0, d0, m0, sh0, ss0, sd0, sc0)
    B1 = (h1, s1, d1, m1, sh1, ss1, sd1, sc1)
    pltpu.sync_copy(sd_hbm.at[pl.ds(w * _CHUNKS, _CHUNKS)], sd_l)

    def _zrow(i, _):
        for v in range(5):
            m0[i, pl.ds(v * 16, 16)] = zeros16
            m1[i, pl.ds(v * 16, 16)] = zeros16
        return 0
    lax.fori_loop(0, _CH, _zrow, 0)
    for k in range(_ZROWS // _CH):
        pltpu.sync_copy(m0, accum.at[pl.ds(s * _ZROWS + k * _CH, _CH)])
    plsc.subcore_barrier()

    def scatter_issue(jl, B):
        hb, sb, db, mb, sh, ss, sd, sc = B
        pltpu.async_copy(mb, accum.at[sd_l.at[jl, 1]], sc, add=True)

    def scatter_wait(jl, B):
        hb, sb, db, mb, sh, ss, sd, sc = B
        pltpu.make_async_copy(mb, accum.at[sd_l.at[jl, 1]], sc).wait()

    def load_issue(jl, B):
        hb, sb, db, mb, sh, ss, sd, sc = B
        pltpu.async_copy(htab_hbm.at[sd_l.at[jl, 0]], hb, sh)
        pltpu.async_copy(asad_hbm.at[sd_l.at[jl, 0]], sb, ss)
        pltpu.async_copy(adas_hbm.at[sd_l.at[jl, 1]], db, sd)

    def process(jl, B):
        hb, sb, db, mb, sh, ss, sd, sc = B
        pltpu.make_async_copy(asad_hbm.at[sd_l.at[jl, 0]], sb, ss).wait()
        pltpu.make_async_copy(adas_hbm.at[sd_l.at[jl, 1]], db, sd).wait()
        pltpu.make_async_copy(htab_hbm.at[sd_l.at[jl, 0]], hb, sh).wait()
        scatter_wait(jl, B)

        @plsc.parallel_loop(0, _CH, 1, unroll=4)
        def _edge(i):
            e = sb[i, :] + db[i, :]
            e = jnp.where(e > 0, e, 0.2 * e)
            ex = jnp.exp(e)
            ex_s[i, :] = ex
            mb[i, pl.ds(64, 16)] = jnp.where(lane_lt8, ex, 0.0)
            bi = iota * 0 + i
            for v in range(2):
                y = hb[i, pl.ds(32 * v, 32)]
                a, b = plsc.unpack(y, format=plsc.PackFormat.INTERLEAVED)
                av = plsc.load_gather(ex_s, [bi, 4 * v + iota // 4])
                mb[i, pl.ds(32 * v, 16)] = a * av
                mb[i, pl.ds(32 * v + 16, 16)] = b * av
        scatter_issue(jl, B)

    _fill_i32(idxp, iota, _N)
    for B in (B0, B1):
        hb, sb, db, mb, sh, ss, sd, sc = B
        pltpu.async_copy(mb, accum.at[idxp], sc, add=True)

    load_issue(0, B0)

    def _pair(t, _):
        j0 = 2 * t
        load_issue(j0 + 1, B1)
        process(j0, B0)
        load_issue(j0 + 2, B0)
        process(j0 + 1, B1)
        return 0
    lax.fori_loop(0, (_CHUNKS - 1) // 2, _pair, 0)
    process(_CHUNKS - 1, B0)
    scatter_wait(_CHUNKS - 1, B0)
    scatter_wait(_CHUNKS - 2, B1)

    plsc.subcore_barrier()
    for k in range(_ZROWS // _CH):
        r0 = s * _ZROWS + k * _CH
        pltpu.sync_copy(accum.at[pl.ds(r0, _CH)], out_hbm.at[c, pl.ds(r0, _CH)])


def _sc_layer2(sd_hbm, tab_hbm, out_hbm,
               sd_l, idxp, s0, s1, d0, d1, m0, m1, ex_s,
               ss0, ss1, sd0, sd1, sc0, sc1, accum):
    c = lax.axis_index("c")
    s = lax.axis_index("s")
    w = c * 16 + s
    iota = lax.iota(jnp.int32, 16)
    zeros16 = jnp.zeros((16,), jnp.float32)
    B0 = (s0, d0, m0, ss0, sd0, sc0)
    B1 = (s1, d1, m1, ss1, sd1, sc1)
    pltpu.sync_copy(sd_hbm.at[pl.ds(w * _CHUNKS, _CHUNKS)], sd_l)

    def _zrow(i, _):
        m0[i, :] = zeros16
        m1[i, :] = zeros16
        return 0
    lax.fori_loop(0, _CH, _zrow, 0)
    for k in range(_ZROWS // _CH):
        pltpu.sync_copy(m0, accum.at[pl.ds(s * _ZROWS + k * _CH, _CH)])
    plsc.subcore_barrier()

    def scatter_issue(jl, B):
        sb, db, mb, ss, sd, sc = B
        pltpu.async_copy(mb, accum.at[sd_l.at[jl, 1]], sc, add=True)

    def scatter_wait(jl, B):
        sb, db, mb, ss, sd, sc = B
        pltpu.make_async_copy(mb, accum.at[sd_l.at[jl, 1]], sc).wait()

    def load_issue(jl, B):
        sb, db, mb, ss, sd, sc = B
        pltpu.async_copy(tab_hbm.at[sd_l.at[jl, 0]], sb, ss)
        pltpu.async_copy(tab_hbm.at[sd_l.at[jl, 1]], db, sd)

    def process(jl, B):
        sb, db, mb, ss, sd, sc = B
        pltpu.make_async_copy(tab_hbm.at[sd_l.at[jl, 0]], sb, ss).wait()
        pltpu.make_async_copy(tab_hbm.at[sd_l.at[jl, 1]], db, sd).wait()
        scatter_wait(jl, B)

        @plsc.parallel_loop(0, _CH // 16, 1, unroll=2)
        def _grp(g):
            rows = g * 16 + iota
            e = (plsc.load_gather(sb, [rows, iota * 0 + 7])
                 + plsc.load_gather(db, [rows, iota * 0 + 8]))
            e = jnp.where(e > 0, e, 0.2 * e)
            ex_s[g, :] = jnp.exp(e)
            bg = iota * 0 + g
            for u in range(16):
                i = g * 16 + u
                srow = sb[i, :]
                exu = plsc.load_gather(ex_s, [bg, iota * 0 + u])
                t = jnp.where(iota == 7, 1.0, srow)
                mb[i, :] = jnp.where(iota < 8, t * exu, 0.0)
        scatter_issue(jl, B)

    _fill_i32(idxp, iota, _N)
    for B in (B0, B1):
        sb, db, mb, ss, sd, sc = B
        pltpu.async_copy(mb, accum.at[idxp], sc, add=True)

    load_issue(0, B0)

    def _pair(t, _):
        j0 = 2 * t
        load_issue(j0 + 1, B1)
        process(j0, B0)
        load_issue(j0 + 2, B0)
        process(j0 + 1, B1)
        return 0
    lax.fori_loop(0, (_CHUNKS - 1) // 2, _pair, 0)
    process(_CHUNKS - 1, B0)
    scatter_wait(_CHUNKS - 1, B0)
    scatter_wait(_CHUNKS - 2, B1)

    plsc.subcore_barrier()
    for k in range(_ZROWS // _CH):
        r0 = s * _ZROWS + k * _CH
        pltpu.sync_copy(accum.at[pl.ds(r0, _CH)], out_hbm.at[c, pl.ds(r0, _CH)])


def _run_sc1(sd, h1tab, asad1, adas1):
    mesh = plsc.VectorSubcoreMesh(core_axis_name="c", subcore_axis_name="s")
    f = pl.kernel(
        _sc_layer1,
        out_type=jax.ShapeDtypeStruct((2, _AROWS, 80), jnp.float32),
        mesh=mesh,
        scratch_types=[
            pltpu.VMEM((_CHUNKS, 2, _CH), jnp.int32),
            pltpu.VMEM((_CH,), jnp.int32),
            pltpu.VMEM((_CH, 64), jnp.bfloat16),
            pltpu.VMEM((_CH, 64), jnp.bfloat16),
            pltpu.VMEM((_CH, 16), jnp.float32),
            pltpu.VMEM((_CH, 16), jnp.float32),
            pltpu.VMEM((_CH, 16), jnp.float32),
            pltpu.VMEM((_CH, 16), jnp.float32),
            pltpu.VMEM((_CH, 80), jnp.float32),
            pltpu.VMEM((_CH, 80), jnp.float32),
            pltpu.VMEM((_CH, 16), jnp.float32),
            pltpu.SemaphoreType.DMA,
            pltpu.SemaphoreType.DMA,
            pltpu.SemaphoreType.DMA,
            pltpu.SemaphoreType.DMA,
            pltpu.SemaphoreType.DMA,
            pltpu.SemaphoreType.DMA,
            pltpu.SemaphoreType.DMA,
            pltpu.SemaphoreType.DMA,
            pltpu.VMEM_SHARED((_AROWS, 80), jnp.float32),
        ],
        compiler_params=pltpu.CompilerParams(
            use_tc_tiling_on_sc=False, needs_layout_passes=False),
    )
    return f(sd, h1tab, asad1, adas1)


def _run_sc2(sd, tab2):
    mesh = plsc.VectorSubcoreMesh(core_axis_name="c", subcore_axis_name="s")
    f = pl.kernel(
        _sc_layer2,
        out_type=jax.ShapeDtypeStruct((2, _AROWS, 16), jnp.float32),
        mesh=mesh,
        scratch_types=[
            pltpu.VMEM((_CHUNKS, 2, _CH), jnp.int32),
            pltpu.VMEM((_CH,), jnp.int32),
            pltpu.VMEM((_CH, 16), jnp.float32),
            pltpu.VMEM((_CH, 16), jnp.float32),
            pltpu.VMEM((_CH, 16), jnp.float32),
            pltpu.VMEM((_CH, 16), jnp.float32),
            pltpu.VMEM((_CH, 16), jnp.float32),
            pltpu.VMEM((_CH, 16), jnp.float32),
            pltpu.VMEM((_CH // 16, 16), jnp.float32),
            pltpu.SemaphoreType.DMA,
            pltpu.SemaphoreType.DMA,
            pltpu.SemaphoreType.DMA,
            pltpu.SemaphoreType.DMA,
            pltpu.SemaphoreType.DMA,
            pltpu.SemaphoreType.DMA,
            pltpu.VMEM_SHARED((_AROWS, 16), jnp.float32),
        ],
        compiler_params=pltpu.CompilerParams(
            use_tc_tiling_on_sc=False, needs_layout_passes=False),
    )
    return f(sd, tab2)


def kernel(x, edge_index, W1, a_src1, a_dst1, b1, W2, a_src2, a_dst2, b2):
    loops = jnp.arange(_N, dtype=jnp.int32)
    src = jnp.concatenate([edge_index[0], loops])
    dst = jnp.concatenate([edge_index[1], loops])
    npad = _EPAD - src.shape[0]
    srcp = jnp.concatenate([src, jnp.zeros((npad,), jnp.int32)])
    dstp = jnp.concatenate([dst, jnp.full((npad,), _N, jnp.int32)])
    sd = jnp.stack([srcp.reshape(-1, _CH), dstp.reshape(-1, _CH)], axis=1)

    eye8 = jnp.eye(8, dtype=jnp.float32)
    a1s = (eye8[:, None, :] * a_src1[:, :, None]).reshape(64, 8)
    a1d = (eye8[:, None, :] * a_dst1[:, :, None]).reshape(64, 8)
    A1 = jnp.concatenate([a1s, a1d], axis=1)
    A1r = jnp.concatenate([a1d, a1s], axis=1)

    perm = jnp.asarray([(j // 32) * 32 + 2 * (j % 16) + ((j // 16) % 2)
                        for j in range(64)], dtype=jnp.int32)
    E64 = jnp.zeros((80, 64), jnp.float32).at[jnp.arange(64), perm].set(1.0)
    BDEN = jnp.concatenate(
        [jnp.zeros((64, 64), jnp.float32),
         jnp.repeat(eye8, 8, axis=1),
         jnp.zeros((8, 64), jnp.float32)], 0)

    M = jnp.zeros((7, 16), jnp.float32)
    M = M.at[:, 0:7].set(jnp.eye(7, dtype=jnp.float32))
    M = M.at[:, 7].set(a_src2[0])
    M = M.at[:, 8].set(a_dst2[0])
    W2M = W2 @ M

    D16 = jnp.zeros((16, 16), jnp.float32).at[7, :].set(1.0)
    b2p = jnp.concatenate([b2, jnp.zeros((9,), jnp.float32)])

    h1tab, asad1, adas1 = pl.pallas_call(
        _tc_prep1,
        out_shape=(jax.ShapeDtypeStruct((_N, 64), jnp.bfloat16),
                   jax.ShapeDtypeStruct((_N, 16), jnp.float32),
                   jax.ShapeDtypeStruct((_N, 16), jnp.float32)),
    )(x, W1, A1, A1r)

    part1 = _run_sc1(sd, h1tab, asad1, adas1)

    tab2 = pl.pallas_call(
        _tc_mid,
        out_shape=jax.ShapeDtypeStruct((_N, 16), jnp.float32),
    )(part1, E64, BDEN, b1, W2M)

    part2 = _run_sc2(sd, tab2)

    out16 = pl.pallas_call(
        _tc_out,
        out_shape=jax.ShapeDtypeStruct((_N, 16), jnp.float32),
    )(part2, D16, b2p)

    return out16[:, :7]


# speedup vs baseline: 197.4936x; 1.0140x over previous
import functools

import jax
import jax.numpy as jnp
from jax import lax
from jax.experimental import pallas as pl
from jax.experimental.pallas import tpu as pltpu
from jax.experimental.pallas import tpu_sc as plsc

_N = 10000
_CH = 128
_CHUNKS = 81
_NW = 32
_EPAD = _NW * _CHUNKS * _CH
_AROWS = 10240
_ZROWS = _AROWS // 16


def _tc_prep1(x_ref, w1_ref, a1_ref, a1r_ref, h_ref, asad_ref, adas_ref):
    h = jnp.dot(x_ref[...], w1_ref[...], preferred_element_type=jnp.float32)
    h_ref[...] = h.astype(jnp.bfloat16)
    asad_ref[...] = jnp.dot(h, a1_ref[...], preferred_element_type=jnp.float32)
    adas_ref[...] = jnp.dot(h, a1r_ref[...], preferred_element_type=jnp.float32)


def _tc_mid(p_ref, e64_ref, bden_ref, b1_ref, w2m_ref, tab_ref):
    pp = p_ref[0] + p_ref[1]
    pp = pp[:_N]
    msg = jnp.dot(pp, e64_ref[...], preferred_element_type=jnp.float32)
    den = jnp.dot(pp, bden_ref[...], preferred_element_type=jnp.float32)
    x1 = msg / den + b1_ref[...]
    x1 = jnp.where(x1 > 0, x1, jnp.exp(jnp.minimum(x1, 0.0)) - 1.0)
    tab_ref[...] = jnp.dot(x1, w2m_ref[...], preferred_element_type=jnp.float32)


def _tc_out(p_ref, d16_ref, b2p_ref, o_ref):
    pp = p_ref[0, :_N] + p_ref[1, :_N]
    den = jnp.dot(pp, d16_ref[...], preferred_element_type=jnp.float32)
    o = pp / den + b2p_ref[...]
    mask = lax.broadcasted_iota(jnp.int32, (1, 16), 1) < 7
    om = jnp.where(mask, o, -1e30)
    m = jnp.max(om, axis=1, keepdims=True)
    ez = jnp.where(mask, jnp.exp(om - m), 0.0)
    lse = jnp.log(jnp.sum(ez, axis=1, keepdims=True)) + m
    o_ref[...] = o - lse


def _fill_i32(ref_row, iota, val):
    for k in range(8):
        ref_row[pl.ds(k * 16, 16)] = iota * 0 + val


def _sc_layer1(sd_hbm, htab_hbm, asad_hbm, adas_hbm, out_hbm,
               sd_l, idxp, h0, h1, s0, s1, d0, d1, m0, m1, ex_s,
               sh0, sh1, ss0, ss1, sd0, sd1, sc0, sc1, accum):
    c = lax.axis_index("c")
    s = lax.axis_index("s")
    w = c * 16 + s
    iota = lax.iota(jnp.int32, 16)
    lane_lt8 = iota < 8
    zeros16 = jnp.zeros((16,), jnp.float32)
    B0 = (h0, s0, d0, m0, sh0, ss0, sd0, sc0)
    B1 = (h1, s1, d1, m1, sh1, ss1, sd1, sc1)
    pltpu.sync_copy(sd_hbm.at[pl.ds(w * _CHUNKS, _CHUNKS)], sd_l)

    def _zrow(i, _):
        for v in range(5):
            m0[i, pl.ds(v * 16, 16)] = zeros16
            m1[i, pl.ds(v * 16, 16)] = zeros16
        return 0
    lax.fori_loop(0, _CH, _zrow, 0)
    for k in range(_ZROWS // _CH):
        pltpu.sync_copy(m0, accum.at[pl.ds(s * _ZROWS + k * _CH, _CH)])
    plsc.subcore_barrier()

    def scatter_issue(jl, B):
        hb, sb, db, mb, sh, ss, sd, sc = B
        pltpu.async_copy(mb, accum.at[sd_l.at[jl, 1]], sc, add=True)

    def scatter_wait(jl, B):
        hb, sb, db, mb, sh, ss, sd, sc = B
        pltpu.make_async_copy(mb, accum.at[sd_l.at[jl, 1]], sc).wait()

    def load_issue(jl, B):
        hb, sb, db, mb, sh, ss, sd, sc = B
        pltpu.async_copy(htab_hbm.at[sd_l.at[jl, 0]], hb, sh)
        pltpu.async_copy(asad_hbm.at[sd_l.at[jl, 0]], sb, ss)
        pltpu.async_copy(adas_hbm.at[sd_l.at[jl, 1]], db, sd)

    def process(jl, B):
        hb, sb, db, mb, sh, ss, sd, sc = B
        pltpu.make_async_copy(asad_hbm.at[sd_l.at[jl, 0]], sb, ss).wait()
        pltpu.make_async_copy(adas_hbm.at[sd_l.at[jl, 1]], db, sd).wait()
        pltpu.make_async_copy(htab_hbm.at[sd_l.at[jl, 0]], hb, sh).wait()
        scatter_wait(jl, B)

        @plsc.parallel_loop(0, _CH, 1, unroll=8)
        def _edge(i):
            e = sb[i, :] + db[i, :]
            e = jnp.where(e > 0, e, 0.2 * e)
            ex = jnp.exp(e)
            ex_s[i, :] = ex
            mb[i, pl.ds(64, 16)] = jnp.where(lane_lt8, ex, 0.0)
            bi = iota * 0 + i
            for v in range(2):
                y = hb[i, pl.ds(32 * v, 32)]
                a, b = plsc.unpack(y, format=plsc.PackFormat.INTERLEAVED)
                av = plsc.load_gather(ex_s, [bi, 4 * v + iota // 4])
                mb[i, pl.ds(32 * v, 16)] = a * av
                mb[i, pl.ds(32 * v + 16, 16)] = b * av
        scatter_issue(jl, B)

    _fill_i32(idxp, iota, _N)
    for B in (B0, B1):
        hb, sb, db, mb, sh, ss, sd, sc = B
        pltpu.async_copy(mb, accum.at[idxp], sc, add=True)

    load_issue(0, B0)

    def _pair(t, _):
        j0 = 2 * t
        load_issue(j0 + 1, B1)
        process(j0, B0)
        load_issue(j0 + 2, B0)
        process(j0 + 1, B1)
        return 0
    lax.fori_loop(0, (_CHUNKS - 1) // 2, _pair, 0)
    process(_CHUNKS - 1, B0)
    scatter_wait(_CHUNKS - 1, B0)
    scatter_wait(_CHUNKS - 2, B1)

    plsc.subcore_barrier()
    for k in range(_ZROWS // _CH):
        r0 = s * _ZROWS + k * _CH
        pltpu.sync_copy(accum.at[pl.ds(r0, _CH)], out_hbm.at[c, pl.ds(r0, _CH)])


def _sc_layer2(sd_hbm, tab_hbm, out_hbm,
               sd_l, idxp, s0, s1, d0, d1, m0, m1, ex_s,
               ss0, ss1, sd0, sd1, sc0, sc1, accum):
    c = lax.axis_index("c")
    s = lax.axis_index("s")
    w = c * 16 + s
    iota = lax.iota(jnp.int32, 16)
    zeros16 = jnp.zeros((16,), jnp.float32)
    B0 = (s0, d0, m0, ss0, sd0, sc0)
    B1 = (s1, d1, m1, ss1, sd1, sc1)
    pltpu.sync_copy(sd_hbm.at[pl.ds(w * _CHUNKS, _CHUNKS)], sd_l)

    def _zrow(i, _):
        m0[i, :] = zeros16
        m1[i, :] = zeros16
        return 0
    lax.fori_loop(0, _CH, _zrow, 0)
    for k in range(_ZROWS // _CH):
        pltpu.sync_copy(m0, accum.at[pl.ds(s * _ZROWS + k * _CH, _CH)])
    plsc.subcore_barrier()

    def scatter_issue(jl, B):
        sb, db, mb, ss, sd, sc = B
        pltpu.async_copy(mb, accum.at[sd_l.at[jl, 1]], sc, add=True)

    def scatter_wait(jl, B):
        sb, db, mb, ss, sd, sc = B
        pltpu.make_async_copy(mb, accum.at[sd_l.at[jl, 1]], sc).wait()

    def load_issue(jl, B):
        sb, db, mb, ss, sd, sc = B
        pltpu.async_copy(tab_hbm.at[sd_l.at[jl, 0]], sb, ss)
        pltpu.async_copy(tab_hbm.at[sd_l.at[jl, 1]], db, sd)

    def process(jl, B):
        sb, db, mb, ss, sd, sc = B
        pltpu.make_async_copy(tab_hbm.at[sd_l.at[jl, 0]], sb, ss).wait()
        pltpu.make_async_copy(tab_hbm.at[sd_l.at[jl, 1]], db, sd).wait()
        scatter_wait(jl, B)

        @plsc.parallel_loop(0, _CH // 16, 1, unroll=4)
        def _grp(g):
            rows = g * 16 + iota
            e = (plsc.load_gather(sb, [rows, iota * 0 + 7])
                 + plsc.load_gather(db, [rows, iota * 0 + 8]))
            e = jnp.where(e > 0, e, 0.2 * e)
            ex_s[g, :] = jnp.exp(e)
            bg = iota * 0 + g
            for u in range(16):
                i = g * 16 + u
                srow = sb[i, :]
                exu = plsc.load_gather(ex_s, [bg, iota * 0 + u])
                t = jnp.where(iota == 7, 1.0, srow)
                mb[i, :] = jnp.where(iota < 8, t * exu, 0.0)
        scatter_issue(jl, B)

    _fill_i32(idxp, iota, _N)
    for B in (B0, B1):
        sb, db, mb, ss, sd, sc = B
        pltpu.async_copy(mb, accum.at[idxp], sc, add=True)

    load_issue(0, B0)

    def _pair(t, _):
        j0 = 2 * t
        load_issue(j0 + 1, B1)
        process(j0, B0)
        load_issue(j0 + 2, B0)
        process(j0 + 1, B1)
        return 0
    lax.fori_loop(0, (_CHUNKS - 1) // 2, _pair, 0)
    process(_CHUNKS - 1, B0)
    scatter_wait(_CHUNKS - 1, B0)
    scatter_wait(_CHUNKS - 2, B1)

    plsc.subcore_barrier()
    for k in range(_ZROWS // _CH):
        r0 = s * _ZROWS + k * _CH
        pltpu.sync_copy(accum.at[pl.ds(r0, _CH)], out_hbm.at[c, pl.ds(r0, _CH)])


def _run_sc1(sd, h1tab, asad1, adas1):
    mesh = plsc.VectorSubcoreMesh(core_axis_name="c", subcore_axis_name="s")
    f = pl.kernel(
        _sc_layer1,
        out_type=jax.ShapeDtypeStruct((2, _AROWS, 80), jnp.float32),
        mesh=mesh,
        scratch_types=[
            pltpu.VMEM((_CHUNKS, 2, _CH), jnp.int32),
            pltpu.VMEM((_CH,), jnp.int32),
            pltpu.VMEM((_CH, 64), jnp.bfloat16),
            pltpu.VMEM((_CH, 64), jnp.bfloat16),
            pltpu.VMEM((_CH, 16), jnp.float32),
            pltpu.VMEM((_CH, 16), jnp.float32),
            pltpu.VMEM((_CH, 16), jnp.float32),
            pltpu.VMEM((_CH, 16), jnp.float32),
            pltpu.VMEM((_CH, 80), jnp.float32),
            pltpu.VMEM((_CH, 80), jnp.float32),
            pltpu.VMEM((_CH, 16), jnp.float32),
            pltpu.SemaphoreType.DMA,
            pltpu.SemaphoreType.DMA,
            pltpu.SemaphoreType.DMA,
            pltpu.SemaphoreType.DMA,
            pltpu.SemaphoreType.DMA,
            pltpu.SemaphoreType.DMA,
            pltpu.SemaphoreType.DMA,
            pltpu.SemaphoreType.DMA,
            pltpu.VMEM_SHARED((_AROWS, 80), jnp.float32),
        ],
        compiler_params=pltpu.CompilerParams(
            use_tc_tiling_on_sc=False, needs_layout_passes=False),
    )
    return f(sd, h1tab, asad1, adas1)


def _run_sc2(sd, tab2):
    mesh = plsc.VectorSubcoreMesh(core_axis_name="c", subcore_axis_name="s")
    f = pl.kernel(
        _sc_layer2,
        out_type=jax.ShapeDtypeStruct((2, _AROWS, 16), jnp.float32),
        mesh=mesh,
        scratch_types=[
            pltpu.VMEM((_CHUNKS, 2, _CH), jnp.int32),
            pltpu.VMEM((_CH,), jnp.int32),
            pltpu.VMEM((_CH, 16), jnp.float32),
            pltpu.VMEM((_CH, 16), jnp.float32),
            pltpu.VMEM((_CH, 16), jnp.float32),
            pltpu.VMEM((_CH, 16), jnp.float32),
            pltpu.VMEM((_CH, 16), jnp.float32),
            pltpu.VMEM((_CH, 16), jnp.float32),
            pltpu.VMEM((_CH // 16, 16), jnp.float32),
            pltpu.SemaphoreType.DMA,
            pltpu.SemaphoreType.DMA,
            pltpu.SemaphoreType.DMA,
            pltpu.SemaphoreType.DMA,
            pltpu.SemaphoreType.DMA,
            pltpu.SemaphoreType.DMA,
            pltpu.VMEM_SHARED((_AROWS, 16), jnp.float32),
        ],
        compiler_params=pltpu.CompilerParams(
            use_tc_tiling_on_sc=False, needs_layout_passes=False),
    )
    return f(sd, tab2)


def kernel(x, edge_index, W1, a_src1, a_dst1, b1, W2, a_src2, a_dst2, b2):
    loops = jnp.arange(_N, dtype=jnp.int32)
    src = jnp.concatenate([edge_index[0], loops])
    dst = jnp.concatenate([edge_index[1], loops])
    npad = _EPAD - src.shape[0]
    srcp = jnp.concatenate([src, jnp.zeros((npad,), jnp.int32)])
    dstp = jnp.concatenate([dst, jnp.full((npad,), _N, jnp.int32)])
    sd = jnp.stack([srcp.reshape(-1, _CH), dstp.reshape(-1, _CH)], axis=1)

    eye8 = jnp.eye(8, dtype=jnp.float32)
    a1s = (eye8[:, None, :] * a_src1[:, :, None]).reshape(64, 8)
    a1d = (eye8[:, None, :] * a_dst1[:, :, None]).reshape(64, 8)
    A1 = jnp.concatenate([a1s, a1d], axis=1)
    A1r = jnp.concatenate([a1d, a1s], axis=1)

    perm = jnp.asarray([(j // 32) * 32 + 2 * (j % 16) + ((j // 16) % 2)
                        for j in range(64)], dtype=jnp.int32)
    E64 = jnp.zeros((80, 64), jnp.float32).at[jnp.arange(64), perm].set(1.0)
    BDEN = jnp.concatenate(
        [jnp.zeros((64, 64), jnp.float32),
         jnp.repeat(eye8, 8, axis=1),
         jnp.zeros((8, 64), jnp.float32)], 0)

    M = jnp.zeros((7, 16), jnp.float32)
    M = M.at[:, 0:7].set(jnp.eye(7, dtype=jnp.float32))
    M = M.at[:, 7].set(a_src2[0])
    M = M.at[:, 8].set(a_dst2[0])
    W2M = W2 @ M

    D16 = jnp.zeros((16, 16), jnp.float32).at[7, :].set(1.0)
    b2p = jnp.concatenate([b2, jnp.zeros((9,), jnp.float32)])

    h1tab, asad1, adas1 = pl.pallas_call(
        _tc_prep1,
        out_shape=(jax.ShapeDtypeStruct((_N, 64), jnp.bfloat16),
                   jax.ShapeDtypeStruct((_N, 16), jnp.float32),
                   jax.ShapeDtypeStruct((_N, 16), jnp.float32)),
    )(x, W1, A1, A1r)

    part1 = _run_sc1(sd, h1tab, asad1, adas1)

    tab2 = pl.pallas_call(
        _tc_mid,
        out_shape=jax.ShapeDtypeStruct((_N, 16), jnp.float32),
    )(part1, E64, BDEN, b1, W2M)

    part2 = _run_sc2(sd, tab2)

    out16 = pl.pallas_call(
        _tc_out,
        out_shape=jax.ShapeDtypeStruct((_N, 16), jnp.float32),
    )(part2, D16, b2p)

    return out16[:, :7]


# speedup vs baseline: 197.6340x; 1.0007x over previous
import jax
import jax.numpy as jnp
from jax import lax
from jax.experimental import pallas as pl
from jax.experimental.pallas import tpu as pltpu
from jax.experimental.pallas import tpu_sc as plsc

_N = 10000
_CH = 128
_CHUNKS = 81
_NW = 32
_EPAD = _NW * _CHUNKS * _CH
_AROWS = 10240
_ZROWS = _AROWS // 16


def _tc_prep1(x_ref, w1_ref, a1_ref, a1r_ref, h_ref, asad_ref, adas_ref):
    h = jnp.dot(x_ref[...], w1_ref[...], preferred_element_type=jnp.float32)
    h_ref[...] = h.astype(jnp.bfloat16)
    asad_ref[...] = jnp.dot(h, a1_ref[...], preferred_element_type=jnp.float32)
    adas_ref[...] = jnp.dot(h, a1r_ref[...], preferred_element_type=jnp.float32)


def _tc_mid(p_ref, e64_ref, bden_ref, b1_ref, w2m_ref, tab_ref):
    pp = p_ref[0] + p_ref[1]
    pp = pp[:_N]
    msg = jnp.dot(pp, e64_ref[...], preferred_element_type=jnp.float32)
    den = jnp.dot(pp, bden_ref[...], preferred_element_type=jnp.float32)
    x1 = msg / den + b1_ref[...]
    x1 = jnp.where(x1 > 0, x1, jnp.exp(jnp.minimum(x1, 0.0)) - 1.0)
    tab_ref[...] = jnp.dot(x1, w2m_ref[...], preferred_element_type=jnp.float32)


def _tc_out(p_ref, d16_ref, b2p_ref, o_ref):
    pp = p_ref[0, :_N] + p_ref[1, :_N]
    den = jnp.dot(pp, d16_ref[...], preferred_element_type=jnp.float32)
    o = pp / den + b2p_ref[...]
    mask = lax.broadcasted_iota(jnp.int32, (1, 16), 1) < 7
    om = jnp.where(mask, o, -1e30)
    m = jnp.max(om, axis=1, keepdims=True)
    ez = jnp.where(mask, jnp.exp(om - m), 0.0)
    lse = jnp.log(jnp.sum(ez, axis=1, keepdims=True)) + m
    o_ref[...] = o - lse


def _fill_i32(ref_row, iota, val):
    for k in range(8):
        ref_row[pl.ds(k * 16, 16)] = iota * 0 + val


def _sc_layer1(sd_hbm, htab_hbm, asad_hbm, adas_hbm, out_hbm,
               sd_l, idxp, h0, h1, s0, s1, d0, d1, m0, m1, ex_s,
               sh0, sh1, ss0, ss1, sd0, sd1, sc0, sc1, accum):
    c = lax.axis_index("c")
    s = lax.axis_index("s")
    w = c * 16 + s
    iota = lax.iota(jnp.int32, 16)
    lane_lt8 = iota < 8
    zeros16 = jnp.zeros((16,), jnp.float32)
    B0 = (h0, s0, d0, m0, sh0, ss0, sd0, sc0)
    B1 = (h1, s1, d1, m1, sh1, ss1, sd1, sc1)
    pltpu.sync_copy(sd_hbm.at[pl.ds(w * _CHUNKS, _CHUNKS)], sd_l)

    def _zrow(i, _):
        for v in range(5):
            m0[i, pl.ds(v * 16, 16)] = zeros16
            m1[i, pl.ds(v * 16, 16)] = zeros16
        return 0
    lax.fori_loop(0, _CH, _zrow, 0)
    for k in range(_ZROWS // _CH):
        pltpu.sync_copy(m0, accum.at[pl.ds(s * _ZROWS + k * _CH, _CH)])
    plsc.subcore_barrier()

    def scatter_issue(jl, B):
        hb, sb, db, mb, sh, ss, sd, sc = B
        pltpu.async_copy(mb, accum.at[sd_l.at[jl, 1]], sc, add=True)

    def scatter_wait(jl, B):
        hb, sb, db, mb, sh, ss, sd, sc = B
        pltpu.make_async_copy(mb, accum.at[sd_l.at[jl, 1]], sc).wait()

    def load_issue(jl, B):
        hb, sb, db, mb, sh, ss, sd, sc = B
        pltpu.async_copy(htab_hbm.at[sd_l.at[jl, 0]], hb, sh)
        pltpu.async_copy(asad_hbm.at[sd_l.at[jl, 0]], sb, ss)
        pltpu.async_copy(adas_hbm.at[sd_l.at[jl, 1]], db, sd)

    def process(jl, B):
        hb, sb, db, mb, sh, ss, sd, sc = B
        pltpu.make_async_copy(asad_hbm.at[sd_l.at[jl, 0]], sb, ss).wait()
        pltpu.make_async_copy(adas_hbm.at[sd_l.at[jl, 1]], db, sd).wait()
        pltpu.make_async_copy(htab_hbm.at[sd_l.at[jl, 0]], hb, sh).wait()
        scatter_wait(jl, B)

        @plsc.parallel_loop(0, _CH, 1, unroll=8)
        def _edge(i):
            e = sb[i, :] + db[i, :]
            e = jnp.where(e > 0, e, 0.2 * e)
            ex = jnp.exp(e)
            ex_s[i, :] = ex
            mb[i, pl.ds(64, 16)] = jnp.where(lane_lt8, ex, 0.0)
            bi = iota * 0 + i
            for v in range(2):
                y = hb[i, pl.ds(32 * v, 32)]
                a, b = plsc.unpack(y, format=plsc.PackFormat.INTERLEAVED)
                av = plsc.load_gather(ex_s, [bi, 4 * v + iota // 4])
                mb[i, pl.ds(32 * v, 16)] = a * av
                mb[i, pl.ds(32 * v + 16, 16)] = b * av
        scatter_issue(jl, B)

    _fill_i32(idxp, iota, _N)
    for B in (B0, B1):
        hb, sb, db, mb, sh, ss, sd, sc = B
        pltpu.async_copy(mb, accum.at[idxp], sc, add=True)

    load_issue(0, B0)

    def _pair(t, _):
        j0 = 2 * t
        load_issue(j0 + 1, B1)
        process(j0, B0)
        load_issue(j0 + 2, B0)
        process(j0 + 1, B1)
        return 0
    lax.fori_loop(0, (_CHUNKS - 1) // 2, _pair, 0)
    process(_CHUNKS - 1, B0)
    scatter_wait(_CHUNKS - 1, B0)
    scatter_wait(_CHUNKS - 2, B1)

    plsc.subcore_barrier()
    for k in range(_ZROWS // _CH):
        r0 = s * _ZROWS + k * _CH
        pltpu.sync_copy(accum.at[pl.ds(r0, _CH)], out_hbm.at[c, pl.ds(r0, _CH)])


def _sc_layer2(sd_hbm, tab_hbm, out_hbm,
               sd_l, idxp, s0, s1, d0, d1, m0, m1, ex_s,
               ss0, ss1, sd0, sd1, sc0, sc1, accum):
    c = lax.axis_index("c")
    s = lax.axis_index("s")
    w = c * 16 + s
    iota = lax.iota(jnp.int32, 16)
    zeros16 = jnp.zeros((16,), jnp.float32)
    B0 = (s0, d0, m0, ss0, sd0, sc0)
    B1 = (s1, d1, m1, ss1, sd1, sc1)
    pltpu.sync_copy(sd_hbm.at[pl.ds(w * _CHUNKS, _CHUNKS)], sd_l)

    def _zrow(i, _):
        m0[i, :] = zeros16
        m1[i, :] = zeros16
        return 0
    lax.fori_loop(0, _CH, _zrow, 0)
    for k in range(_ZROWS // _CH):
        pltpu.sync_copy(m0, accum.at[pl.ds(s * _ZROWS + k * _CH, _CH)])
    plsc.subcore_barrier()

    def scatter_issue(jl, B):
        sb, db, mb, ss, sd, sc = B
        pltpu.async_copy(mb, accum.at[sd_l.at[jl, 1]], sc, add=True)

    def scatter_wait(jl, B):
        sb, db, mb, ss, sd, sc = B
        pltpu.make_async_copy(mb, accum.at[sd_l.at[jl, 1]], sc).wait()

    def load_issue(jl, B):
        sb, db, mb, ss, sd, sc = B
        pltpu.async_copy(tab_hbm.at[sd_l.at[jl, 0]], sb, ss)
        pltpu.async_copy(tab_hbm.at[sd_l.at[jl, 1]], db, sd)

    def process(jl, B):
        sb, db, mb, ss, sd, sc = B
        pltpu.make_async_copy(tab_hbm.at[sd_l.at[jl, 0]], sb, ss).wait()
        pltpu.make_async_copy(tab_hbm.at[sd_l.at[jl, 1]], db, sd).wait()
        scatter_wait(jl, B)

        @plsc.parallel_loop(0, _CH // 16, 1, unroll=4)
        def _grp(g):
            rows = g * 16 + iota
            e = (plsc.load_gather(sb, [rows, iota * 0 + 7])
                 + plsc.load_gather(db, [rows, iota * 0 + 8]))
            e = jnp.where(e > 0, e, 0.2 * e)
            ex_s[g, :] = jnp.exp(e)
            bg = iota * 0 + g
            for u in range(16):
                i = g * 16 + u
                srow = sb[i, :]
                exu = plsc.load_gather(ex_s, [bg, iota * 0 + u])
                t = jnp.where(iota == 7, 1.0, srow)
                mb[i, :] = jnp.where(iota < 8, t * exu, 0.0)
        scatter_issue(jl, B)

    _fill_i32(idxp, iota, _N)
    for B in (B0, B1):
        sb, db, mb, ss, sd, sc = B
        pltpu.async_copy(mb, accum.at[idxp], sc, add=True)

    load_issue(0, B0)

    def _pair(t, _):
        j0 = 2 * t
        load_issue(j0 + 1, B1)
        process(j0, B0)
        load_issue(j0 + 2, B0)
        process(j0 + 1, B1)
        return 0
    lax.fori_loop(0, (_CHUNKS - 1) // 2, _pair, 0)
    process(_CHUNKS - 1, B0)
    scatter_wait(_CHUNKS - 1, B0)
    scatter_wait(_CHUNKS - 2, B1)

    plsc.subcore_barrier()
    for k in range(_ZROWS // _CH):
        r0 = s * _ZROWS + k * _CH
        pltpu.sync_copy(accum.at[pl.ds(r0, _CH)], out_hbm.at[c, pl.ds(r0, _CH)])


def _run_sc1(sd, h1tab, asad1, adas1):
    mesh = plsc.VectorSubcoreMesh(core_axis_name="c", subcore_axis_name="s")
    f = pl.kernel(
        _sc_layer1,
        out_type=jax.ShapeDtypeStruct((2, _AROWS, 80), jnp.float32),
        mesh=mesh,
        scratch_types=[
            pltpu.VMEM((_CHUNKS, 2, _CH), jnp.int32),
            pltpu.VMEM((_CH,), jnp.int32),
            pltpu.VMEM((_CH, 64), jnp.bfloat16),
            pltpu.VMEM((_CH, 64), jnp.bfloat16),
            pltpu.VMEM((_CH, 16), jnp.float32),
            pltpu.VMEM((_CH, 16), jnp.float32),
            pltpu.VMEM((_CH, 16), jnp.float32),
            pltpu.VMEM((_CH, 16), jnp.float32),
            pltpu.VMEM((_CH, 80), jnp.float32),
            pltpu.VMEM((_CH, 80), jnp.float32),
            pltpu.VMEM((_CH, 16), jnp.float32),
            pltpu.SemaphoreType.DMA,
            pltpu.SemaphoreType.DMA,
            pltpu.SemaphoreType.DMA,
            pltpu.SemaphoreType.DMA,
            pltpu.SemaphoreType.DMA,
            pltpu.SemaphoreType.DMA,
            pltpu.SemaphoreType.DMA,
            pltpu.SemaphoreType.DMA,
            pltpu.VMEM_SHARED((_AROWS, 80), jnp.float32),
        ],
        compiler_params=pltpu.CompilerParams(
            use_tc_tiling_on_sc=False, needs_layout_passes=False),
    )
    return f(sd, h1tab, asad1, adas1)


def _run_sc2(sd, tab2):
    mesh = plsc.VectorSubcoreMesh(core_axis_name="c", subcore_axis_name="s")
    f = pl.kernel(
        _sc_layer2,
        out_type=jax.ShapeDtypeStruct((2, _AROWS, 16), jnp.float32),
        mesh=mesh,
        scratch_types=[
            pltpu.VMEM((_CHUNKS, 2, _CH), jnp.int32),
            pltpu.VMEM((_CH,), jnp.int32),
            pltpu.VMEM((_CH, 16), jnp.float32),
            pltpu.VMEM((_CH, 16), jnp.float32),
            pltpu.VMEM((_CH, 16), jnp.float32),
            pltpu.VMEM((_CH, 16), jnp.float32),
            pltpu.VMEM((_CH, 16), jnp.float32),
            pltpu.VMEM((_CH, 16), jnp.float32),
            pltpu.VMEM((_CH // 16, 16), jnp.float32),
            pltpu.SemaphoreType.DMA,
            pltpu.SemaphoreType.DMA,
            pltpu.SemaphoreType.DMA,
            pltpu.SemaphoreType.DMA,
            pltpu.SemaphoreType.DMA,
            pltpu.SemaphoreType.DMA,
            pltpu.VMEM_SHARED((_AROWS, 16), jnp.float32),
        ],
        compiler_params=pltpu.CompilerParams(
            use_tc_tiling_on_sc=False, needs_layout_passes=False),
    )
    return f(sd, tab2)


def kernel(x, edge_index, W1, a_src1, a_dst1, b1, W2, a_src2, a_dst2, b2):
    loops = jnp.arange(_N, dtype=jnp.int32)
    src = jnp.concatenate([edge_index[0], loops])
    dst = jnp.concatenate([edge_index[1], loops])
    npad = _EPAD - src.shape[0]
    srcp = jnp.concatenate([src, jnp.zeros((npad,), jnp.int32)])
    dstp = jnp.concatenate([dst, jnp.full((npad,), _N, jnp.int32)])
    sd = jnp.stack([srcp.reshape(-1, _CH), dstp.reshape(-1, _CH)], axis=1)

    eye8 = jnp.eye(8, dtype=jnp.float32)
    a1s = (eye8[:, None, :] * a_src1[:, :, None]).reshape(64, 8)
    a1d = (eye8[:, None, :] * a_dst1[:, :, None]).reshape(64, 8)
    A1 = jnp.concatenate([a1s, a1d], axis=1)
    A1r = jnp.concatenate([a1d, a1s], axis=1)

    perm = jnp.asarray([(j // 32) * 32 + 2 * (j % 16) + ((j // 16) % 2)
                        for j in range(64)], dtype=jnp.int32)
    E64 = jnp.zeros((80, 64), jnp.float32).at[jnp.arange(64), perm].set(1.0)
    BDEN = jnp.concatenate(
        [jnp.zeros((64, 64), jnp.float32),
         jnp.repeat(eye8, 8, axis=1),
         jnp.zeros((8, 64), jnp.float32)], 0)

    M = jnp.zeros((7, 16), jnp.float32)
    M = M.at[:, 0:7].set(jnp.eye(7, dtype=jnp.float32))
    M = M.at[:, 7].set(a_src2[0])
    M = M.at[:, 8].set(a_dst2[0])
    W2M = W2 @ M

    D16 = jnp.zeros((16, 16), jnp.float32).at[7, :].set(1.0)
    b2p = jnp.concatenate([b2, jnp.zeros((9,), jnp.float32)])

    h1tab, asad1, adas1 = pl.pallas_call(
        _tc_prep1,
        out_shape=(jax.ShapeDtypeStruct((_N, 64), jnp.bfloat16),
                   jax.ShapeDtypeStruct((_N, 16), jnp.float32),
                   jax.ShapeDtypeStruct((_N, 16), jnp.float32)),
    )(x, W1, A1, A1r)

    part1 = _run_sc1(sd, h1tab, asad1, adas1)

    tab2 = pl.pallas_call(
        _tc_mid,
        out_shape=jax.ShapeDtypeStruct((_N, 16), jnp.float32),
    )(part1, E64, BDEN, b1, W2M)

    part2 = _run_sc2(sd, tab2)

    out16 = pl.pallas_call(
        _tc_out,
        out_shape=jax.ShapeDtypeStruct((_N, 16), jnp.float32),
    )(part2, D16, b2p)

    return out16[:, :7]
